# Initial kernel scaffold; baseline (speedup 1.0000x reference)
import jax
import jax.numpy as jnp
from jax.experimental import pallas as pl


def kernel(edge_feat, node_feat, graph_attr, edge_index, We, be, Wn, bn, Wa, ba, Wce, bce, Wcn, bcn, Wca, bca):
    raise NotImplementedError("write your pallas kernel here")


# speedup vs baseline: 3.3358x; 3.3358x over previous
import functools

import jax
import jax.numpy as jnp
from jax import lax
from jax.experimental import pallas as pl
from jax.experimental.pallas import tpu as pltpu
from jax.experimental.pallas import tpu_sc as plsc

N = 10000
E = 320000
D = 128

_NC = 2
_NS = 16
_NW = _NC * _NS
_PER_W = E // _NW
_CH = 80
_NCH = _PER_W // _CH
_NPAD = 10240
_ROWS_PER_TILE = _NPAD // _NS
_ZCH = 128

_BN = 1024
_GN = _NPAD // _BN
_BE = 2560
_GE = E // _BE

_sp = jax.nn.softplus


def _prep_body(v0_ref, wn_ref, bn_ref, w1_ref, w2_ref,
               u0_ref, wa_ref, ba_ref, w4_ref, bce_ref, wcnu_ref, bcn_ref,
               v_ref, a_ref, b_ref, crow_ref, ucn_ref, urow_ref):
    i = pl.program_id(0)
    v = _sp(jnp.dot(v0_ref[...], wn_ref[...], preferred_element_type=jnp.float32)
            + bn_ref[...])
    v_ref[...] = v
    a_ref[...] = jnp.dot(v, w1_ref[...], preferred_element_type=jnp.float32)
    b_ref[...] = jnp.dot(v, w2_ref[...], preferred_element_type=jnp.float32)

    @pl.when(i == 0)
    def _():
        u = _sp(jnp.dot(u0_ref[...], wa_ref[...], preferred_element_type=jnp.float32)
                + ba_ref[...])
        urow_ref[...] = u
        crow_ref[...] = jnp.dot(u, w4_ref[...], preferred_element_type=jnp.float32) + bce_ref[...]
        ucn_ref[...] = jnp.dot(u, wcnu_ref[...], preferred_element_type=jnp.float32) + bcn_ref[...]


def _prep_call(v0, Wn, bn, W1, W2, u0, Wa, ba, W4, bce, WcnU, bcn):
    full = pl.BlockSpec((D, D), lambda i: (0, 0))
    row = pl.BlockSpec((1, D), lambda i: (0, 0))
    blk = pl.BlockSpec((_BN, D), lambda i: (i, 0))
    return pl.pallas_call(
        _prep_body,
        grid=(_GN,),
        in_specs=[blk, full, row, full, full,
                  row, full, row, full, row, full, row],
        out_specs=[blk, blk, blk, row, row, row],
        out_shape=[
            jax.ShapeDtypeStruct((N, D), jnp.float32),
            jax.ShapeDtypeStruct((N, D), jnp.float32),
            jax.ShapeDtypeStruct((N, D), jnp.float32),
            jax.ShapeDtypeStruct((1, D), jnp.float32),
            jax.ShapeDtypeStruct((1, D), jnp.float32),
            jax.ShapeDtypeStruct((1, D), jnp.float32),
        ],
    )(v0, Wn, bn, W1, W2, u0, Wa, ba, W4, bce, WcnU, bcn)


def _sc_gather_body(a_hbm, b_hbm, src_hbm, dst_hbm, gs_hbm, gd_hbm,
                    idx_s, idx_d, bufa, bufb, sema, semb):
    c = lax.axis_index("c")
    s = lax.axis_index("s")
    wid = s * _NC + c
    base = wid * _PER_W

    def body(j, carry):
        off = pl.multiple_of(base + j * _CH, _CH)
        pltpu.sync_copy(src_hbm.at[pl.ds(off, _CH)], idx_s)
        pltpu.sync_copy(dst_hbm.at[pl.ds(off, _CH)], idx_d)
        cpa = pltpu.async_copy(a_hbm.at[idx_s], bufa, sema)
        cpb = pltpu.async_copy(b_hbm.at[idx_d], bufb, semb)
        cpa.wait()
        cpb.wait()
        pltpu.sync_copy(bufa, gs_hbm.at[pl.ds(off, _CH)])
        pltpu.sync_copy(bufb, gd_hbm.at[pl.ds(off, _CH)])
        return carry

    lax.fori_loop(0, _NCH, body, 0)


@functools.partial(
    pl.kernel,
    out_type=[jax.ShapeDtypeStruct((E, D), jnp.float32),
              jax.ShapeDtypeStruct((E, D), jnp.float32)],
    mesh=plsc.VectorSubcoreMesh(core_axis_name="c", subcore_axis_name="s"),
    scratch_types=[
        pltpu.VMEM((_CH,), jnp.int32),
        pltpu.VMEM((_CH,), jnp.int32),
        pltpu.VMEM((_CH, D), jnp.float32),
        pltpu.VMEM((_CH, D), jnp.float32),
        pltpu.SemaphoreType.DMA,
        pltpu.SemaphoreType.DMA,
    ],
)
def _sc_gather(a_hbm, b_hbm, src_hbm, dst_hbm, gs_hbm, gd_hbm,
               idx_s, idx_d, bufa, bufb, sema, semb):
    _sc_gather_body(a_hbm, b_hbm, src_hbm, dst_hbm, gs_hbm, gd_hbm,
                    idx_s, idx_d, bufa, bufb, sema, semb)


def _sc_cnt_body(dst_hbm, cnt_hbm, idx_all, tab):
    c = lax.axis_index("c")
    s = lax.axis_index("s")
    wid = s * _NC + c
    base = wid * _PER_W
    zero16 = jnp.zeros((16,), jnp.float32)
    one16 = jnp.ones((16,), jnp.float32)

    def zfill(r, carry):
        tab[pl.ds(r * 16, 16)] = zero16
        return carry

    lax.fori_loop(0, _NPAD // 16, zfill, 0)

    pltpu.sync_copy(dst_hbm.at[pl.ds(base, _PER_W)], idx_all)

    def body(i, carry):
        ids = idx_all[pl.ds(i * 16, 16)]
        plsc.addupdate_scatter(tab, [ids], one16)
        return carry

    lax.fori_loop(0, _PER_W // 16, body, 0)
    pltpu.sync_copy(tab, cnt_hbm.at[wid])


@functools.partial(
    pl.kernel,
    out_type=jax.ShapeDtypeStruct((_NW, _NPAD), jnp.float32),
    mesh=plsc.VectorSubcoreMesh(core_axis_name="c", subcore_axis_name="s"),
    scratch_types=[
        pltpu.VMEM((_PER_W,), jnp.int32),
        pltpu.VMEM((_NPAD,), jnp.float32),
    ],
    compiler_params=pltpu.CompilerParams(needs_layout_passes=False),
)
def _sc_cnt(dst_hbm, cnt_hbm, idx_all, tab):
    _sc_cnt_body(dst_hbm, cnt_hbm, idx_all, tab)


def _edge_body(e0_ref, gs_ref, gd_ref, we_ref, be_ref, w3_ref, crow_ref,
               oute_ref, enew_ref, ue_ref):
    i = pl.program_id(0)
    e0 = e0_ref[...]
    e = _sp(jnp.dot(e0, we_ref[...], preferred_element_type=jnp.float32) + be_ref[...])
    t = jnp.dot(e, w3_ref[...], preferred_element_type=jnp.float32)
    en = _sp(t + gs_ref[...] + gd_ref[...] + crow_ref[...])
    oute_ref[...] = en + e0
    enew_ref[...] = en
    part = jnp.sum(en.reshape(_BE // 8, 8, D), axis=0)

    @pl.when(i == 0)
    def _():
        ue_ref[...] = part

    @pl.when(i > 0)
    def _():
        ue_ref[...] += part


def _edge_call(e0, Gs, Gd, We, be, W3, crow):
    blk = pl.BlockSpec((_BE, D), lambda i: (i, 0))
    full = pl.BlockSpec((D, D), lambda i: (0, 0))
    row = pl.BlockSpec((1, D), lambda i: (0, 0))
    return pl.pallas_call(
        _edge_body,
        grid=(_GE,),
        in_specs=[blk, blk, blk, full, row, full, row],
        out_specs=[blk, blk, pl.BlockSpec((8, D), lambda i: (0, 0))],
        out_shape=[
            jax.ShapeDtypeStruct((E, D), jnp.float32),
            jax.ShapeDtypeStruct((E, D), jnp.float32),
            jax.ShapeDtypeStruct((8, D), jnp.float32),
        ],
    )(e0, Gs, Gd, We, be, W3, crow)


def _sc_scatter_body(enew_hbm, dst_hbm, esum_hbm,
                     idx_d, rows, zbuf, acc, sem):
    c = lax.axis_index("c")
    s = lax.axis_index("s")
    wid = s * _NC + c
    base = wid * _PER_W

    zero16 = jnp.zeros((16,), jnp.float32)

    def zfill(r, carry):
        for cc in range(D // 16):
            zbuf[r, pl.ds(cc * 16, 16)] = zero16
        return carry

    lax.fori_loop(0, _ZCH, zfill, 0)

    for k in range(_ROWS_PER_TILE // _ZCH):
        r0 = s * _ROWS_PER_TILE + k * _ZCH
        pltpu.sync_copy(zbuf, acc.at[pl.ds(r0, _ZCH)])
    plsc.subcore_barrier()

    def body(j, carry):
        off = pl.multiple_of(base + j * _CH, _CH)
        pltpu.sync_copy(dst_hbm.at[pl.ds(off, _CH)], idx_d)
        pltpu.async_copy(enew_hbm.at[pl.ds(off, _CH)], rows, sem).wait()
        pltpu.sync_copy(rows, acc.at[idx_d], add=True)
        return carry

    lax.fori_loop(0, _NCH, body, 0)
    plsc.subcore_barrier()

    for k in range(_ROWS_PER_TILE // _ZCH):
        r0 = s * _ROWS_PER_TILE + k * _ZCH
        pltpu.sync_copy(acc.at[pl.ds(r0, _ZCH)], esum_hbm.at[c, pl.ds(r0, _ZCH)])


@functools.partial(
    pl.kernel,
    out_type=jax.ShapeDtypeStruct((_NC, _NPAD, D), jnp.float32),
    mesh=plsc.VectorSubcoreMesh(core_axis_name="c", subcore_axis_name="s"),
    scratch_types=[
        pltpu.VMEM((_CH,), jnp.int32),
        pltpu.VMEM((_CH, D), jnp.float32),
        pltpu.VMEM((_ZCH, D), jnp.float32),
        pltpu.VMEM_SHARED((_NPAD, D), jnp.float32),
        pltpu.SemaphoreType.DMA,
    ],
)
def _sc_scatter(enew_hbm, dst_hbm, esum_hbm,
                idx_d, rows, zbuf, acc, sem):
    _sc_scatter_body(enew_hbm, dst_hbm, esum_hbm,
                     idx_d, rows, zbuf, acc, sem)


def _node_body(v_ref, v0_ref, es0_ref, es1_ref, cnt_ref,
               ucn_ref, wv_ref, wve_ref,
               urow_ref, ue_ref, wa1_ref, wa2_ref, wa3_ref, bca_ref, u0_ref,
               outv_ref, outu_ref, uvacc_ref):
    i = pl.program_id(0)
    es = es0_ref[...] + es1_ref[...]
    cnt = jnp.sum(jnp.transpose(cnt_ref[...]), axis=1, keepdims=True)
    ve = es / jnp.maximum(cnt, 1.0)
    vn = _sp(jnp.dot(v_ref[...], wv_ref[...], preferred_element_type=jnp.float32)
             + jnp.dot(ve, wve_ref[...], preferred_element_type=jnp.float32)
             + ucn_ref[...])
    outv_ref[...] = vn + v0_ref[...]
    rows = i * _BN + lax.broadcasted_iota(jnp.int32, (_BN, 1), 0)
    vn_masked = jnp.where(rows < N, vn, 0.0)
    part = jnp.sum(vn_masked.reshape(_BN // 8, 8, D), axis=0)

    @pl.when(i == 0)
    def _():
        uvacc_ref[...] = part

    @pl.when(i > 0)
    def _():
        uvacc_ref[...] += part

    @pl.when(i == _GN - 1)
    def _():
        uv = jnp.sum(uvacc_ref[...], axis=0, keepdims=True) * (1.0 / N)
        ue = jnp.sum(ue_ref[...], axis=0, keepdims=True) * (1.0 / E)
        un = _sp(jnp.dot(urow_ref[...], wa1_ref[...], preferred_element_type=jnp.float32)
                 + jnp.dot(ue, wa2_ref[...], preferred_element_type=jnp.float32)
                 + jnp.dot(uv, wa3_ref[...], preferred_element_type=jnp.float32)
                 + bca_ref[...])
        outu_ref[...] = un + u0_ref[...]


def _node_call(v, v0, es0, es1, cnt_all, ucn, WcnV, WcnE,
               urow, ue_part, Wa1, Wa2, Wa3, bca, u0):
    blk = pl.BlockSpec((_BN, D), lambda i: (i, 0))
    cblk = pl.BlockSpec((_NW, _BN), lambda i: (0, i))
    full = pl.BlockSpec((D, D), lambda i: (0, 0))
    row = pl.BlockSpec((1, D), lambda i: (0, 0))
    return pl.pallas_call(
        _node_body,
        grid=(_GN,),
        in_specs=[blk, blk, blk, blk, cblk,
                  row, full, full,
                  row, pl.BlockSpec((8, D), lambda i: (0, 0)),
                  full, full, full, row, row],
        out_specs=[blk, row],
        out_shape=[
            jax.ShapeDtypeStruct((N, D), jnp.float32),
            jax.ShapeDtypeStruct((1, D), jnp.float32),
        ],
        scratch_shapes=[pltpu.VMEM((8, D), jnp.float32)],
    )(v, v0, es0, es1, cnt_all, ucn, WcnV, WcnE,
      urow, ue_part, Wa1, Wa2, Wa3, bca, u0)


def kernel(edge_feat, node_feat, graph_attr, edge_index,
           We, be, Wn, bn, Wa, ba, Wce, bce, Wcn, bcn, Wca, bca):
    src = edge_index[0]
    dst = edge_index[1]
    be_r = be.reshape(1, D)
    bn_r = bn.reshape(1, D)
    ba_r = ba.reshape(1, D)
    bce_r = bce.reshape(1, D)
    bcn_r = bcn.reshape(1, D)
    bca_r = bca.reshape(1, D)
    W1, W2, W3, W4 = Wce[:D], Wce[D:2 * D], Wce[2 * D:3 * D], Wce[3 * D:]
    WcnV, WcnE, WcnU = Wcn[:D], Wcn[D:2 * D], Wcn[2 * D:]
    Wa1, Wa2, Wa3 = Wca[:D], Wca[D:2 * D], Wca[2 * D:]

    v, A, B, crow, ucn, urow = _prep_call(
        node_feat, Wn, bn_r, W1, W2, graph_attr, Wa, ba_r, W4, bce_r, WcnU, bcn_r)

    Gs, Gd = _sc_gather(A, B, src, dst)
    cnt_all = _sc_cnt(dst)

    out_e, e_new, ue_part = _edge_call(edge_feat, Gs, Gd, We, be_r, W3, crow)

    esum_part = _sc_scatter(e_new, dst)

    out_v, out_u = _node_call(
        v, node_feat, esum_part[0], esum_part[1], cnt_all,
        ucn, WcnV, WcnE, urow, ue_part, Wa1, Wa2, Wa3, bca_r, graph_attr)

    return (out_e, out_v, out_u)


# speedup vs baseline: 4.7184x; 1.4145x over previous
import functools

import jax
import jax.numpy as jnp
from jax import lax
from jax.experimental import pallas as pl
from jax.experimental.pallas import tpu as pltpu
from jax.experimental.pallas import tpu_sc as plsc

N = 10000
E = 320000
D = 128

_NC = 2
_NS = 16
_NW = _NC * _NS
_PER_W = E // _NW
_CH = 80
_NCH = _PER_W // _CH
_NPAD = 10240
_ROWS_PER_TILE = _NPAD // _NS
_ZCH = 128

_BN = 1024
_GN = _NPAD // _BN
_BE = 2560
_GE = E // _BE

_sp = jax.nn.softplus


def _prep_body(v0_ref, wn_ref, bn_ref, w1_ref, w2_ref,
               u0_ref, wa_ref, ba_ref, w4_ref, bce_ref, wcnu_ref, bcn_ref,
               v_ref, a_ref, b_ref, crow_ref, ucn_ref, urow_ref):
    i = pl.program_id(0)
    v = _sp(jnp.dot(v0_ref[...], wn_ref[...], preferred_element_type=jnp.float32)
            + bn_ref[...])
    v_ref[...] = v
    a_ref[...] = jnp.dot(v, w1_ref[...], preferred_element_type=jnp.float32)
    b_ref[...] = jnp.dot(v, w2_ref[...], preferred_element_type=jnp.float32)

    @pl.when(i == 0)
    def _():
        u = _sp(jnp.dot(u0_ref[...], wa_ref[...], preferred_element_type=jnp.float32)
                + ba_ref[...])
        urow_ref[...] = u
        crow_ref[...] = jnp.dot(u, w4_ref[...], preferred_element_type=jnp.float32) + bce_ref[...]
        ucn_ref[...] = jnp.dot(u, wcnu_ref[...], preferred_element_type=jnp.float32) + bcn_ref[...]


def _prep_call(v0, Wn, bn, W1, W2, u0, Wa, ba, W4, bce, WcnU, bcn):
    full = pl.BlockSpec((D, D), lambda i: (0, 0))
    row = pl.BlockSpec((1, D), lambda i: (0, 0))
    blk = pl.BlockSpec((_BN, D), lambda i: (i, 0))
    return pl.pallas_call(
        _prep_body,
        grid=(_GN,),
        in_specs=[blk, full, row, full, full,
                  row, full, row, full, row, full, row],
        out_specs=[blk, blk, blk, row, row, row],
        out_shape=[
            jax.ShapeDtypeStruct((N, D), jnp.float32),
            jax.ShapeDtypeStruct((N, D), jnp.float32),
            jax.ShapeDtypeStruct((N, D), jnp.float32),
            jax.ShapeDtypeStruct((1, D), jnp.float32),
            jax.ShapeDtypeStruct((1, D), jnp.float32),
            jax.ShapeDtypeStruct((1, D), jnp.float32),
        ],
    )(v0, Wn, bn, W1, W2, u0, Wa, ba, W4, bce, WcnU, bcn)


_K = 5
_SUP = _NCH // _K


def _sc_gather_body(a_hbm, b_hbm, src_hbm, dst_hbm, gs_hbm, gd_hbm,
                    idx_s5, idx_d5, bufa5, bufb5, semi, sema, semw):
    c = lax.axis_index("c")
    s = lax.axis_index("s")
    wid = s * _NC + c
    base = wid * _PER_W

    def body(t, carry):
        off0 = pl.multiple_of(base + t * (_K * _CH), _CH)
        ic = []
        for k in range(_K):
            off = off0 + k * _CH
            ic.append(pltpu.async_copy(src_hbm.at[pl.ds(off, _CH)], idx_s5.at[k], semi))
            ic.append(pltpu.async_copy(dst_hbm.at[pl.ds(off, _CH)], idx_d5.at[k], semi))
        for cp in ic:
            cp.wait()
        gc = []
        for k in range(_K):
            gc.append(pltpu.async_copy(a_hbm.at[idx_s5.at[k]], bufa5.at[k], sema))
            gc.append(pltpu.async_copy(b_hbm.at[idx_d5.at[k]], bufb5.at[k], sema))
        for cp in gc:
            cp.wait()
        wb = []
        for k in range(_K):
            off = off0 + k * _CH
            wb.append(pltpu.async_copy(bufa5.at[k], gs_hbm.at[pl.ds(off, _CH)], semw))
            wb.append(pltpu.async_copy(bufb5.at[k], gd_hbm.at[pl.ds(off, _CH)], semw))
        for cp in wb:
            cp.wait()
        return carry

    lax.fori_loop(0, _SUP, body, 0)


@functools.partial(
    pl.kernel,
    out_type=[jax.ShapeDtypeStruct((E, D), jnp.float32),
              jax.ShapeDtypeStruct((E, D), jnp.float32)],
    mesh=plsc.VectorSubcoreMesh(core_axis_name="c", subcore_axis_name="s"),
    scratch_types=[
        pltpu.VMEM((_K, _CH), jnp.int32),
        pltpu.VMEM((_K, _CH), jnp.int32),
        pltpu.VMEM((_K, _CH, D), jnp.float32),
        pltpu.VMEM((_K, _CH, D), jnp.float32),
        pltpu.SemaphoreType.DMA,
        pltpu.SemaphoreType.DMA,
        pltpu.SemaphoreType.DMA,
    ],
)
def _sc_gather(a_hbm, b_hbm, src_hbm, dst_hbm, gs_hbm, gd_hbm,
               idx_s5, idx_d5, bufa5, bufb5, semi, sema, semw):
    _sc_gather_body(a_hbm, b_hbm, src_hbm, dst_hbm, gs_hbm, gd_hbm,
                    idx_s5, idx_d5, bufa5, bufb5, semi, sema, semw)


def _sc_cnt_body(dst_hbm, cnt_hbm, idx_all, tab):
    c = lax.axis_index("c")
    s = lax.axis_index("s")
    wid = s * _NC + c
    base = wid * _PER_W
    zero16 = jnp.zeros((16,), jnp.float32)
    one16 = jnp.ones((16,), jnp.float32)

    def zfill(r, carry):
        tab[pl.ds(r * 16, 16)] = zero16
        return carry

    lax.fori_loop(0, _NPAD // 16, zfill, 0)

    pltpu.sync_copy(dst_hbm.at[pl.ds(base, _PER_W)], idx_all)

    def body(i, carry):
        ids = idx_all[pl.ds(i * 16, 16)]
        plsc.addupdate_scatter(tab, [ids], one16)
        return carry

    lax.fori_loop(0, _PER_W // 16, body, 0)
    pltpu.sync_copy(tab, cnt_hbm.at[wid])


@functools.partial(
    pl.kernel,
    out_type=jax.ShapeDtypeStruct((_NW, _NPAD), jnp.float32),
    mesh=plsc.VectorSubcoreMesh(core_axis_name="c", subcore_axis_name="s"),
    scratch_types=[
        pltpu.VMEM((_PER_W,), jnp.int32),
        pltpu.VMEM((_NPAD,), jnp.float32),
    ],
    compiler_params=pltpu.CompilerParams(needs_layout_passes=False),
)
def _sc_cnt(dst_hbm, cnt_hbm, idx_all, tab):
    _sc_cnt_body(dst_hbm, cnt_hbm, idx_all, tab)


def _edge_body(e0_ref, gs_ref, gd_ref, we_ref, be_ref, w3_ref, crow_ref,
               oute_ref, enew_ref, ue_ref):
    i = pl.program_id(0)
    e0 = e0_ref[...]
    e = _sp(jnp.dot(e0, we_ref[...], preferred_element_type=jnp.float32) + be_ref[...])
    t = jnp.dot(e, w3_ref[...], preferred_element_type=jnp.float32)
    en = _sp(t + gs_ref[...] + gd_ref[...] + crow_ref[...])
    oute_ref[...] = en + e0
    enew_ref[...] = en
    part = jnp.sum(en.reshape(_BE // 8, 8, D), axis=0)

    @pl.when(i == 0)
    def _():
        ue_ref[...] = part

    @pl.when(i > 0)
    def _():
        ue_ref[...] += part


def _edge_call(e0, Gs, Gd, We, be, W3, crow):
    blk = pl.BlockSpec((_BE, D), lambda i: (i, 0))
    full = pl.BlockSpec((D, D), lambda i: (0, 0))
    row = pl.BlockSpec((1, D), lambda i: (0, 0))
    return pl.pallas_call(
        _edge_body,
        grid=(_GE,),
        in_specs=[blk, blk, blk, full, row, full, row],
        out_specs=[blk, blk, pl.BlockSpec((8, D), lambda i: (0, 0))],
        out_shape=[
            jax.ShapeDtypeStruct((E, D), jnp.float32),
            jax.ShapeDtypeStruct((E, D), jnp.float32),
            jax.ShapeDtypeStruct((8, D), jnp.float32),
        ],
    )(e0, Gs, Gd, We, be, W3, crow)


_SSK = 2
_SCC = 40
_SCH = _SSK * _SCC
_SSUP = _PER_W // _SCH


def _sc_scatter_body(enew_hbm, dst_hbm, esum_hbm,
                     idx0, idx1, rows0, rows1, acc,
                     seml0, seml1, sems):
    c = lax.axis_index("c")
    s = lax.axis_index("s")
    wid = s * _NC + c
    base = wid * _PER_W

    zero16 = jnp.zeros((16,), jnp.float32)

    def zfill(r, carry):
        for cc in range(D // 16):
            rows0[r, pl.ds(cc * 16, 16)] = zero16
        return carry

    lax.fori_loop(0, _SCH, zfill, 0)
    for k in range(_ROWS_PER_TILE // _SCH):
        r0 = s * _ROWS_PER_TILE + k * _SCH
        pltpu.sync_copy(rows0, acc.at[pl.ds(r0, _SCH)])
    plsc.subcore_barrier()

    def issue_loads(sc, idx_b, rows_b, sem):
        off0 = pl.multiple_of(base + sc * _SCH, _SCC)
        for k in range(_SSK):
            pltpu.async_copy(dst_hbm.at[pl.ds(off0 + k * _SCC, _SCC)],
                             idx_b.at[k], sem)
        pltpu.async_copy(enew_hbm.at[pl.ds(off0, _SCH)], rows_b, sem)

    def wait_loads(idx_b, rows_b, sem):
        for k in range(_SSK):
            pltpu.make_async_copy(dst_hbm.at[pl.ds(base, _SCC)],
                                  idx_b.at[k], sem).wait()
        pltpu.make_async_copy(enew_hbm.at[pl.ds(base, _SCH)], rows_b, sem).wait()

    def do_scatter(idx_b, rows_b):
        cps = []
        for k in range(_SSK):
            cps.append(pltpu.async_copy(rows_b.at[pl.ds(k * _SCC, _SCC)],
                                        acc.at[idx_b.at[k]], sems, add=True))
        for cp in cps:
            cp.wait()

    issue_loads(0, idx0, rows0, seml0)

    def body(i, carry):
        issue_loads(2 * i + 1, idx1, rows1, seml1)
        wait_loads(idx0, rows0, seml0)
        do_scatter(idx0, rows0)
        issue_loads(2 * i + 2, idx0, rows0, seml0)
        wait_loads(idx1, rows1, seml1)
        do_scatter(idx1, rows1)
        return carry

    lax.fori_loop(0, (_SSUP - 1) // 2, body, 0)
    wait_loads(idx0, rows0, seml0)
    do_scatter(idx0, rows0)
    plsc.subcore_barrier()

    for k in range(_ROWS_PER_TILE // _ZCH):
        r0 = s * _ROWS_PER_TILE + k * _ZCH
        pltpu.sync_copy(acc.at[pl.ds(r0, _ZCH)], esum_hbm.at[c, pl.ds(r0, _ZCH)])


@functools.partial(
    pl.kernel,
    out_type=jax.ShapeDtypeStruct((_NC, _NPAD, D), jnp.float32),
    mesh=plsc.VectorSubcoreMesh(core_axis_name="c", subcore_axis_name="s"),
    scratch_types=[
        pltpu.VMEM((_SSK, _SCC), jnp.int32),
        pltpu.VMEM((_SSK, _SCC), jnp.int32),
        pltpu.VMEM((_SCH, D), jnp.float32),
        pltpu.VMEM((_SCH, D), jnp.float32),
        pltpu.VMEM_SHARED((_NPAD, D), jnp.float32),
        pltpu.SemaphoreType.DMA,
        pltpu.SemaphoreType.DMA,
        pltpu.SemaphoreType.DMA,
    ],
)
def _sc_scatter(enew_hbm, dst_hbm, esum_hbm,
                idx0, idx1, rows0, rows1, acc, seml0, seml1, sems):
    _sc_scatter_body(enew_hbm, dst_hbm, esum_hbm,
                     idx0, idx1, rows0, rows1, acc, seml0, seml1, sems)


def _node_body(v_ref, v0_ref, es0_ref, es1_ref, cnt_ref,
               ucn_ref, wv_ref, wve_ref,
               urow_ref, ue_ref, wa1_ref, wa2_ref, wa3_ref, bca_ref, u0_ref,
               outv_ref, outu_ref, uvacc_ref):
    i = pl.program_id(0)
    es = es0_ref[...] + es1_ref[...]
    cnt = jnp.sum(jnp.transpose(cnt_ref[...]), axis=1, keepdims=True)
    ve = es / jnp.maximum(cnt, 1.0)
    vn = _sp(jnp.dot(v_ref[...], wv_ref[...], preferred_element_type=jnp.float32)
             + jnp.dot(ve, wve_ref[...], preferred_element_type=jnp.float32)
             + ucn_ref[...])
    outv_ref[...] = vn + v0_ref[...]
    rows = i * _BN + lax.broadcasted_iota(jnp.int32, (_BN, 1), 0)
    vn_masked = jnp.where(rows < N, vn, 0.0)
    part = jnp.sum(vn_masked.reshape(_BN // 8, 8, D), axis=0)

    @pl.when(i == 0)
    def _():
        uvacc_ref[...] = part

    @pl.when(i > 0)
    def _():
        uvacc_ref[...] += part

    @pl.when(i == _GN - 1)
    def _():
        uv = jnp.sum(uvacc_ref[...], axis=0, keepdims=True) * (1.0 / N)
        ue = jnp.sum(ue_ref[...], axis=0, keepdims=True) * (1.0 / E)
        un = _sp(jnp.dot(urow_ref[...], wa1_ref[...], preferred_element_type=jnp.float32)
                 + jnp.dot(ue, wa2_ref[...], preferred_element_type=jnp.float32)
                 + jnp.dot(uv, wa3_ref[...], preferred_element_type=jnp.float32)
                 + bca_ref[...])
        outu_ref[...] = un + u0_ref[...]


def _node_call(v, v0, es0, es1, cnt_all, ucn, WcnV, WcnE,
               urow, ue_part, Wa1, Wa2, Wa3, bca, u0):
    blk = pl.BlockSpec((_BN, D), lambda i: (i, 0))
    cblk = pl.BlockSpec((_NW, _BN), lambda i: (0, i))
    full = pl.BlockSpec((D, D), lambda i: (0, 0))
    row = pl.BlockSpec((1, D), lambda i: (0, 0))
    return pl.pallas_call(
        _node_body,
        grid=(_GN,),
        in_specs=[blk, blk, blk, blk, cblk,
                  row, full, full,
                  row, pl.BlockSpec((8, D), lambda i: (0, 0)),
                  full, full, full, row, row],
        out_specs=[blk, row],
        out_shape=[
            jax.ShapeDtypeStruct((N, D), jnp.float32),
            jax.ShapeDtypeStruct((1, D), jnp.float32),
        ],
        scratch_shapes=[pltpu.VMEM((8, D), jnp.float32)],
    )(v, v0, es0, es1, cnt_all, ucn, WcnV, WcnE,
      urow, ue_part, Wa1, Wa2, Wa3, bca, u0)


def kernel(edge_feat, node_feat, graph_attr, edge_index,
           We, be, Wn, bn, Wa, ba, Wce, bce, Wcn, bcn, Wca, bca):
    src = edge_index[0]
    dst = edge_index[1]
    be_r = be.reshape(1, D)
    bn_r = bn.reshape(1, D)
    ba_r = ba.reshape(1, D)
    bce_r = bce.reshape(1, D)
    bcn_r = bcn.reshape(1, D)
    bca_r = bca.reshape(1, D)
    W1, W2, W3, W4 = Wce[:D], Wce[D:2 * D], Wce[2 * D:3 * D], Wce[3 * D:]
    WcnV, WcnE, WcnU = Wcn[:D], Wcn[D:2 * D], Wcn[2 * D:]
    Wa1, Wa2, Wa3 = Wca[:D], Wca[D:2 * D], Wca[2 * D:]

    v, A, B, crow, ucn, urow = _prep_call(
        node_feat, Wn, bn_r, W1, W2, graph_attr, Wa, ba_r, W4, bce_r, WcnU, bcn_r)

    Gs, Gd = _sc_gather(A, B, src, dst)
    cnt_all = _sc_cnt(dst)

    out_e, e_new, ue_part = _edge_call(edge_feat, Gs, Gd, We, be_r, W3, crow)

    esum_part = _sc_scatter(e_new, dst)

    out_v, out_u = _node_call(
        v, node_feat, esum_part[0], esum_part[1], cnt_all,
        ucn, WcnV, WcnE, urow, ue_part, Wa1, Wa2, Wa3, bca_r, graph_attr)

    return (out_e, out_v, out_u)


# speedup vs baseline: 4.7437x; 1.0054x over previous
import functools

import jax
import jax.numpy as jnp
from jax import lax
from jax.experimental import pallas as pl
from jax.experimental.pallas import tpu as pltpu
from jax.experimental.pallas import tpu_sc as plsc

N = 10000
E = 320000
D = 128

_NC = 2
_NS = 16
_NW = _NC * _NS
_PER_W = E // _NW
_CH = 80
_NCH = _PER_W // _CH
_NPAD = 10240
_ROWS_PER_TILE = _NPAD // _NS
_ZCH = 128

_BN = 1024
_GN = _NPAD // _BN
_BE = 2560
_GE = E // _BE

_sp = jax.nn.softplus


def _prep_body(v0_ref, wn_ref, bn_ref, w1_ref, w2_ref,
               u0_ref, wa_ref, ba_ref, w4_ref, bce_ref, wcnu_ref, bcn_ref,
               v_ref, a_ref, b_ref, crow_ref, ucn_ref, urow_ref):
    i = pl.program_id(0)
    v = _sp(jnp.dot(v0_ref[...], wn_ref[...], preferred_element_type=jnp.float32)
            + bn_ref[...])
    v_ref[...] = v
    a_ref[...] = jnp.dot(v, w1_ref[...], preferred_element_type=jnp.float32)
    b_ref[...] = jnp.dot(v, w2_ref[...], preferred_element_type=jnp.float32)

    @pl.when(i == 0)
    def _():
        u = _sp(jnp.dot(u0_ref[...], wa_ref[...], preferred_element_type=jnp.float32)
                + ba_ref[...])
        urow_ref[...] = u
        crow_ref[...] = jnp.dot(u, w4_ref[...], preferred_element_type=jnp.float32) + bce_ref[...]
        ucn_ref[...] = jnp.dot(u, wcnu_ref[...], preferred_element_type=jnp.float32) + bcn_ref[...]


def _prep_call(v0, Wn, bn, W1, W2, u0, Wa, ba, W4, bce, WcnU, bcn):
    full = pl.BlockSpec((D, D), lambda i: (0, 0))
    row = pl.BlockSpec((1, D), lambda i: (0, 0))
    blk = pl.BlockSpec((_BN, D), lambda i: (i, 0))
    return pl.pallas_call(
        _prep_body,
        grid=(_GN,),
        in_specs=[blk, full, row, full, full,
                  row, full, row, full, row, full, row],
        out_specs=[blk, blk, blk, row, row, row],
        out_shape=[
            jax.ShapeDtypeStruct((N, D), jnp.float32),
            jax.ShapeDtypeStruct((N, D), jnp.float32),
            jax.ShapeDtypeStruct((N, D), jnp.float32),
            jax.ShapeDtypeStruct((1, D), jnp.float32),
            jax.ShapeDtypeStruct((1, D), jnp.float32),
            jax.ShapeDtypeStruct((1, D), jnp.float32),
        ],
    )(v0, Wn, bn, W1, W2, u0, Wa, ba, W4, bce, WcnU, bcn)


_K = 5
_SUP = _NCH // _K


def _sc_gather_body(a_hbm, b_hbm, src_hbm, dst_hbm, gs_hbm, gd_hbm, cnt_hbm,
                    idx_s5, idx_d5, bufa5, bufb5, tab, semi, sema, semw):
    c = lax.axis_index("c")
    s = lax.axis_index("s")
    wid = s * _NC + c
    base = wid * _PER_W

    zero16 = jnp.zeros((16,), jnp.float32)
    one16 = jnp.ones((16,), jnp.float32)

    def zfill(r, carry):
        tab[pl.ds(r * 16, 16)] = zero16
        return carry

    lax.fori_loop(0, _NPAD // 16, zfill, 0)

    def body(t, carry):
        off0 = pl.multiple_of(base + t * (_K * _CH), _CH)
        ic = []
        for k in range(_K):
            off = off0 + k * _CH
            ic.append(pltpu.async_copy(src_hbm.at[pl.ds(off, _CH)], idx_s5.at[k], semi))
            ic.append(pltpu.async_copy(dst_hbm.at[pl.ds(off, _CH)], idx_d5.at[k], semi))
        for cp in ic:
            cp.wait()
        gc = []
        for k in range(_K):
            gc.append(pltpu.async_copy(a_hbm.at[idx_s5.at[k]], bufa5.at[k], sema))
            gc.append(pltpu.async_copy(b_hbm.at[idx_d5.at[k]], bufb5.at[k], sema))
        for cp in gc:
            cp.wait()
        wb = []
        for k in range(_K):
            off = off0 + k * _CH
            wb.append(pltpu.async_copy(bufa5.at[k], gs_hbm.at[pl.ds(off, _CH)], semw))
            wb.append(pltpu.async_copy(bufb5.at[k], gd_hbm.at[pl.ds(off, _CH)], semw))
        for k in range(_K):
            for t16 in range(_CH // 16):
                ids = idx_d5[k, pl.ds(t16 * 16, 16)]
                plsc.addupdate_scatter(tab, [ids], one16)
        for cp in wb:
            cp.wait()
        return carry

    lax.fori_loop(0, _SUP, body, 0)
    pltpu.sync_copy(tab, cnt_hbm.at[wid])


@functools.partial(
    pl.kernel,
    out_type=[jax.ShapeDtypeStruct((E, D), jnp.float32),
              jax.ShapeDtypeStruct((E, D), jnp.float32),
              jax.ShapeDtypeStruct((_NW, _NPAD), jnp.float32)],
    mesh=plsc.VectorSubcoreMesh(core_axis_name="c", subcore_axis_name="s"),
    scratch_types=[
        pltpu.VMEM((_K, _CH), jnp.int32),
        pltpu.VMEM((_K, _CH), jnp.int32),
        pltpu.VMEM((_K, _CH, D), jnp.float32),
        pltpu.VMEM((_K, _CH, D), jnp.float32),
        pltpu.VMEM((_NPAD,), jnp.float32),
        pltpu.SemaphoreType.DMA,
        pltpu.SemaphoreType.DMA,
        pltpu.SemaphoreType.DMA,
    ],
    compiler_params=pltpu.CompilerParams(needs_layout_passes=False),
)
def _sc_gather(a_hbm, b_hbm, src_hbm, dst_hbm, gs_hbm, gd_hbm, cnt_hbm,
               idx_s5, idx_d5, bufa5, bufb5, tab, semi, sema, semw):
    _sc_gather_body(a_hbm, b_hbm, src_hbm, dst_hbm, gs_hbm, gd_hbm, cnt_hbm,
                    idx_s5, idx_d5, bufa5, bufb5, tab, semi, sema, semw)


def _edge_body(e0_ref, gs_ref, gd_ref, we_ref, be_ref, w3_ref, crow_ref,
               oute_ref, enew_ref, ue_ref):
    i = pl.program_id(0)
    e0 = e0_ref[...]
    e = _sp(jnp.dot(e0, we_ref[...], preferred_element_type=jnp.float32) + be_ref[...])
    t = jnp.dot(e, w3_ref[...], preferred_element_type=jnp.float32)
    en = _sp(t + gs_ref[...] + gd_ref[...] + crow_ref[...])
    oute_ref[...] = en + e0
    enew_ref[...] = en
    part = jnp.sum(en.reshape(_BE // 8, 8, D), axis=0)

    @pl.when(i == 0)
    def _():
        ue_ref[...] = part

    @pl.when(i > 0)
    def _():
        ue_ref[...] += part


def _edge_call(e0, Gs, Gd, We, be, W3, crow):
    blk = pl.BlockSpec((_BE, D), lambda i: (i, 0))
    full = pl.BlockSpec((D, D), lambda i: (0, 0))
    row = pl.BlockSpec((1, D), lambda i: (0, 0))
    return pl.pallas_call(
        _edge_body,
        grid=(_GE,),
        in_specs=[blk, blk, blk, full, row, full, row],
        out_specs=[blk, blk, pl.BlockSpec((8, D), lambda i: (0, 0))],
        out_shape=[
            jax.ShapeDtypeStruct((E, D), jnp.float32),
            jax.ShapeDtypeStruct((E, D), jnp.float32),
            jax.ShapeDtypeStruct((8, D), jnp.float32),
        ],
    )(e0, Gs, Gd, We, be, W3, crow)


_SSK = 2
_SCC = 40
_SCH = _SSK * _SCC
_SSUP = _PER_W // _SCH


def _sc_scatter_body(enew_hbm, dst_hbm, esum_hbm,
                     idx0, idx1, rows0, rows1, acc,
                     seml0, seml1, sems):
    c = lax.axis_index("c")
    s = lax.axis_index("s")
    wid = s * _NC + c
    base = wid * _PER_W

    zero16 = jnp.zeros((16,), jnp.float32)

    def zfill(r, carry):
        for cc in range(D // 16):
            rows0[r, pl.ds(cc * 16, 16)] = zero16
        return carry

    lax.fori_loop(0, _SCH, zfill, 0)
    for k in range(_ROWS_PER_TILE // _SCH):
        r0 = s * _ROWS_PER_TILE + k * _SCH
        pltpu.sync_copy(rows0, acc.at[pl.ds(r0, _SCH)])
    plsc.subcore_barrier()

    def issue_loads(sc, idx_b, rows_b, sem):
        off0 = pl.multiple_of(base + sc * _SCH, _SCC)
        for k in range(_SSK):
            pltpu.async_copy(dst_hbm.at[pl.ds(off0 + k * _SCC, _SCC)],
                             idx_b.at[k], sem)
        pltpu.async_copy(enew_hbm.at[pl.ds(off0, _SCH)], rows_b, sem)

    def wait_loads(idx_b, rows_b, sem):
        for k in range(_SSK):
            pltpu.make_async_copy(dst_hbm.at[pl.ds(base, _SCC)],
                                  idx_b.at[k], sem).wait()
        pltpu.make_async_copy(enew_hbm.at[pl.ds(base, _SCH)], rows_b, sem).wait()

    def do_scatter(idx_b, rows_b):
        cps = []
        for k in range(_SSK):
            cps.append(pltpu.async_copy(rows_b.at[pl.ds(k * _SCC, _SCC)],
                                        acc.at[idx_b.at[k]], sems, add=True))
        for cp in cps:
            cp.wait()

    issue_loads(0, idx0, rows0, seml0)

    def body(i, carry):
        issue_loads(2 * i + 1, idx1, rows1, seml1)
        wait_loads(idx0, rows0, seml0)
        do_scatter(idx0, rows0)
        issue_loads(2 * i + 2, idx0, rows0, seml0)
        wait_loads(idx1, rows1, seml1)
        do_scatter(idx1, rows1)
        return carry

    lax.fori_loop(0, (_SSUP - 1) // 2, body, 0)
    wait_loads(idx0, rows0, seml0)
    do_scatter(idx0, rows0)
    plsc.subcore_barrier()

    for k in range(_ROWS_PER_TILE // _ZCH):
        r0 = s * _ROWS_PER_TILE + k * _ZCH
        pltpu.sync_copy(acc.at[pl.ds(r0, _ZCH)], esum_hbm.at[c, pl.ds(r0, _ZCH)])


@functools.partial(
    pl.kernel,
    out_type=jax.ShapeDtypeStruct((_NC, _NPAD, D), jnp.float32),
    mesh=plsc.VectorSubcoreMesh(core_axis_name="c", subcore_axis_name="s"),
    scratch_types=[
        pltpu.VMEM((_SSK, _SCC), jnp.int32),
        pltpu.VMEM((_SSK, _SCC), jnp.int32),
        pltpu.VMEM((_SCH, D), jnp.float32),
        pltpu.VMEM((_SCH, D), jnp.float32),
        pltpu.VMEM_SHARED((_NPAD, D), jnp.float32),
        pltpu.SemaphoreType.DMA,
        pltpu.SemaphoreType.DMA,
        pltpu.SemaphoreType.DMA,
    ],
)
def _sc_scatter(enew_hbm, dst_hbm, esum_hbm,
                idx0, idx1, rows0, rows1, acc, seml0, seml1, sems):
    _sc_scatter_body(enew_hbm, dst_hbm, esum_hbm,
                     idx0, idx1, rows0, rows1, acc, seml0, seml1, sems)


def _node_body(v_ref, v0_ref, es0_ref, es1_ref, cnt_ref,
               ucn_ref, wv_ref, wve_ref,
               urow_ref, ue_ref, wa1_ref, wa2_ref, wa3_ref, bca_ref, u0_ref,
               outv_ref, outu_ref, uvacc_ref):
    i = pl.program_id(0)
    es = es0_ref[...] + es1_ref[...]
    cnt = jnp.sum(jnp.transpose(cnt_ref[...]), axis=1, keepdims=True)
    ve = es / jnp.maximum(cnt, 1.0)
    vn = _sp(jnp.dot(v_ref[...], wv_ref[...], preferred_element_type=jnp.float32)
             + jnp.dot(ve, wve_ref[...], preferred_element_type=jnp.float32)
             + ucn_ref[...])
    outv_ref[...] = vn + v0_ref[...]
    rows = i * _BN + lax.broadcasted_iota(jnp.int32, (_BN, 1), 0)
    vn_masked = jnp.where(rows < N, vn, 0.0)
    part = jnp.sum(vn_masked.reshape(_BN // 8, 8, D), axis=0)

    @pl.when(i == 0)
    def _():
        uvacc_ref[...] = part

    @pl.when(i > 0)
    def _():
        uvacc_ref[...] += part

    @pl.when(i == _GN - 1)
    def _():
        uv = jnp.sum(uvacc_ref[...], axis=0, keepdims=True) * (1.0 / N)
        ue = jnp.sum(ue_ref[...], axis=0, keepdims=True) * (1.0 / E)
        un = _sp(jnp.dot(urow_ref[...], wa1_ref[...], preferred_element_type=jnp.float32)
                 + jnp.dot(ue, wa2_ref[...], preferred_element_type=jnp.float32)
                 + jnp.dot(uv, wa3_ref[...], preferred_element_type=jnp.float32)
                 + bca_ref[...])
        outu_ref[...] = un + u0_ref[...]


def _node_call(v, v0, es0, es1, cnt_all, ucn, WcnV, WcnE,
               urow, ue_part, Wa1, Wa2, Wa3, bca, u0):
    blk = pl.BlockSpec((_BN, D), lambda i: (i, 0))
    cblk = pl.BlockSpec((_NW, _BN), lambda i: (0, i))
    full = pl.BlockSpec((D, D), lambda i: (0, 0))
    row = pl.BlockSpec((1, D), lambda i: (0, 0))
    return pl.pallas_call(
        _node_body,
        grid=(_GN,),
        in_specs=[blk, blk, blk, blk, cblk,
                  row, full, full,
                  row, pl.BlockSpec((8, D), lambda i: (0, 0)),
                  full, full, full, row, row],
        out_specs=[blk, row],
        out_shape=[
            jax.ShapeDtypeStruct((N, D), jnp.float32),
            jax.ShapeDtypeStruct((1, D), jnp.float32),
        ],
        scratch_shapes=[pltpu.VMEM((8, D), jnp.float32)],
    )(v, v0, es0, es1, cnt_all, ucn, WcnV, WcnE,
      urow, ue_part, Wa1, Wa2, Wa3, bca, u0)


def kernel(edge_feat, node_feat, graph_attr, edge_index,
           We, be, Wn, bn, Wa, ba, Wce, bce, Wcn, bcn, Wca, bca):
    src = edge_index[0]
    dst = edge_index[1]
    be_r = be.reshape(1, D)
    bn_r = bn.reshape(1, D)
    ba_r = ba.reshape(1, D)
    bce_r = bce.reshape(1, D)
    bcn_r = bcn.reshape(1, D)
    bca_r = bca.reshape(1, D)
    W1, W2, W3, W4 = Wce[:D], Wce[D:2 * D], Wce[2 * D:3 * D], Wce[3 * D:]
    WcnV, WcnE, WcnU = Wcn[:D], Wcn[D:2 * D], Wcn[2 * D:]
    Wa1, Wa2, Wa3 = Wca[:D], Wca[D:2 * D], Wca[2 * D:]

    v, A, B, crow, ucn, urow = _prep_call(
        node_feat, Wn, bn_r, W1, W2, graph_attr, Wa, ba_r, W4, bce_r, WcnU, bcn_r)

    Gs, Gd, cnt_all = _sc_gather(A, B, src, dst)

    out_e, e_new, ue_part = _edge_call(edge_feat, Gs, Gd, We, be_r, W3, crow)

    esum_part = _sc_scatter(e_new, dst)

    out_v, out_u = _node_call(
        v, node_feat, esum_part[0], esum_part[1], cnt_all,
        ucn, WcnV, WcnE, urow, ue_part, Wa1, Wa2, Wa3, bca_r, graph_attr)

    return (out_e, out_v, out_u)


# speedup vs baseline: 5.2950x; 1.1162x over previous
import functools

import jax
import jax.numpy as jnp
from jax import lax
from jax.experimental import pallas as pl
from jax.experimental.pallas import tpu as pltpu
from jax.experimental.pallas import tpu_sc as plsc

N = 10000
E = 320000
D = 128

_NC = 2
_NS = 16
_NW = _NC * _NS
_PER_W = E // _NW
_CH = 80
_NCH = _PER_W // _CH
_NPAD = 10240
_ROWS_PER_TILE = _NPAD // _NS
_ZCH = 128

_BN = 1024
_GN = _NPAD // _BN
_BE = 2560
_GE = E // _BE

_sp = jax.nn.softplus


def _prep_body(v0_ref, wn_ref, bn_ref, w1_ref, w2_ref,
               u0_ref, wa_ref, ba_ref, w4_ref, bce_ref, wcnu_ref, bcn_ref,
               v_ref, a_ref, b_ref, crow_ref, ucn_ref, urow_ref):
    i = pl.program_id(0)
    v = _sp(jnp.dot(v0_ref[...], wn_ref[...], preferred_element_type=jnp.float32)
            + bn_ref[...])
    v_ref[...] = v
    a_ref[...] = jnp.dot(v, w1_ref[...], preferred_element_type=jnp.float32)
    b_ref[...] = jnp.dot(v, w2_ref[...], preferred_element_type=jnp.float32)

    @pl.when(i == 0)
    def _():
        u = _sp(jnp.dot(u0_ref[...], wa_ref[...], preferred_element_type=jnp.float32)
                + ba_ref[...])
        urow_ref[...] = u
        crow_ref[...] = jnp.dot(u, w4_ref[...], preferred_element_type=jnp.float32) + bce_ref[...]
        ucn_ref[...] = jnp.dot(u, wcnu_ref[...], preferred_element_type=jnp.float32) + bcn_ref[...]


def _prep_call(v0, Wn, bn, W1, W2, u0, Wa, ba, W4, bce, WcnU, bcn):
    full = pl.BlockSpec((D, D), lambda i: (0, 0))
    row = pl.BlockSpec((1, D), lambda i: (0, 0))
    blk = pl.BlockSpec((_BN, D), lambda i: (i, 0))
    return pl.pallas_call(
        _prep_body,
        grid=(_GN,),
        in_specs=[blk, full, row, full, full,
                  row, full, row, full, row, full, row],
        out_specs=[blk, blk, blk, row, row, row],
        out_shape=[
            jax.ShapeDtypeStruct((N, D), jnp.float32),
            jax.ShapeDtypeStruct((N, D), jnp.float32),
            jax.ShapeDtypeStruct((N, D), jnp.float32),
            jax.ShapeDtypeStruct((1, D), jnp.float32),
            jax.ShapeDtypeStruct((1, D), jnp.float32),
            jax.ShapeDtypeStruct((1, D), jnp.float32),
        ],
    )(v0, Wn, bn, W1, W2, u0, Wa, ba, W4, bce, WcnU, bcn)


_K = 5
_SUP = _NCH // _K


def _sc_gather_body(a_hbm, b_hbm, src_hbm, dst_hbm, g_hbm, cnt_hbm,
                    idx_s5, idx_d5, bufa5, bufb5, tab, semi, sema, semw):
    c = lax.axis_index("c")
    s = lax.axis_index("s")
    wid = s * _NC + c
    base = wid * _PER_W

    zero16 = jnp.zeros((16,), jnp.float32)
    one16 = jnp.ones((16,), jnp.float32)

    def zfill(r, carry):
        tab[pl.ds(r * 16, 16)] = zero16
        return carry

    lax.fori_loop(0, _NPAD // 16, zfill, 0)

    def body(t, carry):
        off0 = pl.multiple_of(base + t * (_K * _CH), _CH)
        ic = []
        for k in range(_K):
            off = off0 + k * _CH
            ic.append(pltpu.async_copy(src_hbm.at[pl.ds(off, _CH)], idx_s5.at[k], semi))
            ic.append(pltpu.async_copy(dst_hbm.at[pl.ds(off, _CH)], idx_d5.at[k], semi))
        for cp in ic:
            cp.wait()
        gc = []
        for k in range(_K):
            gc.append(pltpu.async_copy(a_hbm.at[idx_s5.at[k]], bufa5.at[k], sema))
            gc.append(pltpu.async_copy(b_hbm.at[idx_d5.at[k]], bufb5.at[k], sema))
        wb = []
        for k in range(_K):
            gc[2 * k].wait()
            gc[2 * k + 1].wait()

            def addrow(r, carry, _k=k):
                for cc in range(D // 16):
                    sl = pl.ds(cc * 16, 16)
                    bufa5[_k, r, sl] += bufb5[_k, r, sl]
                return carry

            lax.fori_loop(0, _CH, addrow, 0)
            off = off0 + k * _CH
            wb.append(pltpu.async_copy(bufa5.at[k], g_hbm.at[pl.ds(off, _CH)], semw))
        for k in range(_K):
            for t16 in range(_CH // 16):
                ids = idx_d5[k, pl.ds(t16 * 16, 16)]
                plsc.addupdate_scatter(tab, [ids], one16)
        for cp in wb:
            cp.wait()
        return carry

    lax.fori_loop(0, _SUP, body, 0)
    pltpu.sync_copy(tab, cnt_hbm.at[wid])


@functools.partial(
    pl.kernel,
    out_type=[jax.ShapeDtypeStruct((E, D), jnp.float32),
              jax.ShapeDtypeStruct((_NW, _NPAD), jnp.float32)],
    mesh=plsc.VectorSubcoreMesh(core_axis_name="c", subcore_axis_name="s"),
    scratch_types=[
        pltpu.VMEM((_K, _CH), jnp.int32),
        pltpu.VMEM((_K, _CH), jnp.int32),
        pltpu.VMEM((_K, _CH, D), jnp.float32),
        pltpu.VMEM((_K, _CH, D), jnp.float32),
        pltpu.VMEM((_NPAD,), jnp.float32),
        pltpu.SemaphoreType.DMA,
        pltpu.SemaphoreType.DMA,
        pltpu.SemaphoreType.DMA,
    ],
    compiler_params=pltpu.CompilerParams(needs_layout_passes=False),
)
def _sc_gather(a_hbm, b_hbm, src_hbm, dst_hbm, g_hbm, cnt_hbm,
               idx_s5, idx_d5, bufa5, bufb5, tab, semi, sema, semw):
    _sc_gather_body(a_hbm, b_hbm, src_hbm, dst_hbm, g_hbm, cnt_hbm,
                    idx_s5, idx_d5, bufa5, bufb5, tab, semi, sema, semw)


def _edge_body(e0_ref, g_ref, we_ref, be_ref, w3_ref, crow_ref,
               oute_ref, enew_ref, ue_ref):
    i = pl.program_id(0)
    e0 = e0_ref[...]
    e = _sp(jnp.dot(e0, we_ref[...], preferred_element_type=jnp.float32) + be_ref[...])
    t = jnp.dot(e, w3_ref[...], preferred_element_type=jnp.float32)
    en = _sp(t + g_ref[...] + crow_ref[...])
    oute_ref[...] = en + e0
    enew_ref[...] = en
    part = jnp.sum(en.reshape(_BE // 8, 8, D), axis=0)

    @pl.when(i == 0)
    def _():
        ue_ref[...] = part

    @pl.when(i > 0)
    def _():
        ue_ref[...] += part


def _edge_call(e0, G, We, be, W3, crow):
    blk = pl.BlockSpec((_BE, D), lambda i: (i, 0))
    full = pl.BlockSpec((D, D), lambda i: (0, 0))
    row = pl.BlockSpec((1, D), lambda i: (0, 0))
    return pl.pallas_call(
        _edge_body,
        grid=(_GE,),
        in_specs=[blk, blk, full, row, full, row],
        out_specs=[blk, blk, pl.BlockSpec((8, D), lambda i: (0, 0))],
        out_shape=[
            jax.ShapeDtypeStruct((E, D), jnp.float32),
            jax.ShapeDtypeStruct((E, D), jnp.float32),
            jax.ShapeDtypeStruct((8, D), jnp.float32),
        ],
    )(e0, G, We, be, W3, crow)


_SSK = 2
_SCC = 40
_SCH = _SSK * _SCC
_SSUP = _PER_W // _SCH


def _sc_scatter_body(enew_hbm, dst_hbm, esum_hbm,
                     idx0, idx1, rows0, rows1, acc,
                     seml0, seml1, sems):
    c = lax.axis_index("c")
    s = lax.axis_index("s")
    wid = s * _NC + c
    base = wid * _PER_W

    zero16 = jnp.zeros((16,), jnp.float32)

    def zfill(r, carry):
        for cc in range(D // 16):
            rows0[r, pl.ds(cc * 16, 16)] = zero16
        return carry

    lax.fori_loop(0, _SCH, zfill, 0)
    for k in range(_ROWS_PER_TILE // _SCH):
        r0 = s * _ROWS_PER_TILE + k * _SCH
        pltpu.sync_copy(rows0, acc.at[pl.ds(r0, _SCH)])
    plsc.subcore_barrier()

    def issue_loads(sc, idx_b, rows_b, sem):
        off0 = pl.multiple_of(base + sc * _SCH, _SCC)
        for k in range(_SSK):
            pltpu.async_copy(dst_hbm.at[pl.ds(off0 + k * _SCC, _SCC)],
                             idx_b.at[k], sem)
        pltpu.async_copy(enew_hbm.at[pl.ds(off0, _SCH)], rows_b, sem)

    def wait_loads(idx_b, rows_b, sem):
        for k in range(_SSK):
            pltpu.make_async_copy(dst_hbm.at[pl.ds(base, _SCC)],
                                  idx_b.at[k], sem).wait()
        pltpu.make_async_copy(enew_hbm.at[pl.ds(base, _SCH)], rows_b, sem).wait()

    def do_scatter(idx_b, rows_b):
        cps = []
        for k in range(_SSK):
            cps.append(pltpu.async_copy(rows_b.at[pl.ds(k * _SCC, _SCC)],
                                        acc.at[idx_b.at[k]], sems, add=True))
        for cp in cps:
            cp.wait()

    issue_loads(0, idx0, rows0, seml0)

    def body(i, carry):
        issue_loads(2 * i + 1, idx1, rows1, seml1)
        wait_loads(idx0, rows0, seml0)
        do_scatter(idx0, rows0)
        issue_loads(2 * i + 2, idx0, rows0, seml0)
        wait_loads(idx1, rows1, seml1)
        do_scatter(idx1, rows1)
        return carry

    lax.fori_loop(0, (_SSUP - 1) // 2, body, 0)
    wait_loads(idx0, rows0, seml0)
    do_scatter(idx0, rows0)
    plsc.subcore_barrier()

    for k in range(_ROWS_PER_TILE // _ZCH):
        r0 = s * _ROWS_PER_TILE + k * _ZCH
        pltpu.sync_copy(acc.at[pl.ds(r0, _ZCH)], esum_hbm.at[c, pl.ds(r0, _ZCH)])


@functools.partial(
    pl.kernel,
    out_type=jax.ShapeDtypeStruct((_NC, _NPAD, D), jnp.float32),
    mesh=plsc.VectorSubcoreMesh(core_axis_name="c", subcore_axis_name="s"),
    scratch_types=[
        pltpu.VMEM((_SSK, _SCC), jnp.int32),
        pltpu.VMEM((_SSK, _SCC), jnp.int32),
        pltpu.VMEM((_SCH, D), jnp.float32),
        pltpu.VMEM((_SCH, D), jnp.float32),
        pltpu.VMEM_SHARED((_NPAD, D), jnp.float32),
        pltpu.SemaphoreType.DMA,
        pltpu.SemaphoreType.DMA,
        pltpu.SemaphoreType.DMA,
    ],
)
def _sc_scatter(enew_hbm, dst_hbm, esum_hbm,
                idx0, idx1, rows0, rows1, acc, seml0, seml1, sems):
    _sc_scatter_body(enew_hbm, dst_hbm, esum_hbm,
                     idx0, idx1, rows0, rows1, acc, seml0, seml1, sems)


def _node_body(v_ref, v0_ref, es0_ref, es1_ref, cnt_ref,
               ucn_ref, wv_ref, wve_ref,
               urow_ref, ue_ref, wa1_ref, wa2_ref, wa3_ref, bca_ref, u0_ref,
               outv_ref, outu_ref, uvacc_ref):
    i = pl.program_id(0)
    es = es0_ref[...] + es1_ref[...]
    cnt = jnp.sum(jnp.transpose(cnt_ref[...]), axis=1, keepdims=True)
    ve = es / jnp.maximum(cnt, 1.0)
    vn = _sp(jnp.dot(v_ref[...], wv_ref[...], preferred_element_type=jnp.float32)
             + jnp.dot(ve, wve_ref[...], preferred_element_type=jnp.float32)
             + ucn_ref[...])
    outv_ref[...] = vn + v0_ref[...]
    rows = i * _BN + lax.broadcasted_iota(jnp.int32, (_BN, 1), 0)
    vn_masked = jnp.where(rows < N, vn, 0.0)
    part = jnp.sum(vn_masked.reshape(_BN // 8, 8, D), axis=0)

    @pl.when(i == 0)
    def _():
        uvacc_ref[...] = part

    @pl.when(i > 0)
    def _():
        uvacc_ref[...] += part

    @pl.when(i == _GN - 1)
    def _():
        uv = jnp.sum(uvacc_ref[...], axis=0, keepdims=True) * (1.0 / N)
        ue = jnp.sum(ue_ref[...], axis=0, keepdims=True) * (1.0 / E)
        un = _sp(jnp.dot(urow_ref[...], wa1_ref[...], preferred_element_type=jnp.float32)
                 + jnp.dot(ue, wa2_ref[...], preferred_element_type=jnp.float32)
                 + jnp.dot(uv, wa3_ref[...], preferred_element_type=jnp.float32)
                 + bca_ref[...])
        outu_ref[...] = un + u0_ref[...]


def _node_call(v, v0, es0, es1, cnt_all, ucn, WcnV, WcnE,
               urow, ue_part, Wa1, Wa2, Wa3, bca, u0):
    blk = pl.BlockSpec((_BN, D), lambda i: (i, 0))
    cblk = pl.BlockSpec((_NW, _BN), lambda i: (0, i))
    full = pl.BlockSpec((D, D), lambda i: (0, 0))
    row = pl.BlockSpec((1, D), lambda i: (0, 0))
    return pl.pallas_call(
        _node_body,
        grid=(_GN,),
        in_specs=[blk, blk, blk, blk, cblk,
                  row, full, full,
                  row, pl.BlockSpec((8, D), lambda i: (0, 0)),
                  full, full, full, row, row],
        out_specs=[blk, row],
        out_shape=[
            jax.ShapeDtypeStruct((N, D), jnp.float32),
            jax.ShapeDtypeStruct((1, D), jnp.float32),
        ],
        scratch_shapes=[pltpu.VMEM((8, D), jnp.float32)],
    )(v, v0, es0, es1, cnt_all, ucn, WcnV, WcnE,
      urow, ue_part, Wa1, Wa2, Wa3, bca, u0)


def kernel(edge_feat, node_feat, graph_attr, edge_index,
           We, be, Wn, bn, Wa, ba, Wce, bce, Wcn, bcn, Wca, bca):
    src = edge_index[0]
    dst = edge_index[1]
    be_r = be.reshape(1, D)
    bn_r = bn.reshape(1, D)
    ba_r = ba.reshape(1, D)
    bce_r = bce.reshape(1, D)
    bcn_r = bcn.reshape(1, D)
    bca_r = bca.reshape(1, D)
    W1, W2, W3, W4 = Wce[:D], Wce[D:2 * D], Wce[2 * D:3 * D], Wce[3 * D:]
    WcnV, WcnE, WcnU = Wcn[:D], Wcn[D:2 * D], Wcn[2 * D:]
    Wa1, Wa2, Wa3 = Wca[:D], Wca[D:2 * D], Wca[2 * D:]

    v, A, B, crow, ucn, urow = _prep_call(
        node_feat, Wn, bn_r, W1, W2, graph_attr, Wa, ba_r, W4, bce_r, WcnU, bcn_r)

    G, cnt_all = _sc_gather(A, B, src, dst)

    out_e, e_new, ue_part = _edge_call(edge_feat, G, We, be_r, W3, crow)

    esum_part = _sc_scatter(e_new, dst)

    out_v, out_u = _node_call(
        v, node_feat, esum_part[0], esum_part[1], cnt_all,
        ucn, WcnV, WcnE, urow, ue_part, Wa1, Wa2, Wa3, bca_r, graph_attr)

    return (out_e, out_v, out_u)


# speedup vs baseline: 5.3345x; 1.0075x over previous
import functools

import jax
import jax.numpy as jnp
from jax import lax
from jax.experimental import pallas as pl
from jax.experimental.pallas import tpu as pltpu
from jax.experimental.pallas import tpu_sc as plsc

N = 10000
E = 320000
D = 128

_NC = 2
_NS = 16
_NW = _NC * _NS
_PER_W = E // _NW
_CH = 80
_NCH = _PER_W // _CH
_NPAD = 10240
_ROWS_PER_TILE = _NPAD // _NS
_ZCH = 128

_BN = 1024
_GN = _NPAD // _BN
_BE = 2560
_GE = E // _BE

_sp = jax.nn.softplus


def _prep_body(v0_ref, wn_ref, bn_ref, w1_ref, w2_ref,
               u0_ref, wa_ref, ba_ref, w4_ref, bce_ref, wcnu_ref, bcn_ref,
               v_ref, a_ref, b_ref, crow_ref, ucn_ref, urow_ref):
    i = pl.program_id(0)
    v = _sp(jnp.dot(v0_ref[...], wn_ref[...], preferred_element_type=jnp.float32)
            + bn_ref[...])
    v_ref[...] = v
    a_ref[...] = jnp.dot(v, w1_ref[...], preferred_element_type=jnp.float32)
    b_ref[...] = jnp.dot(v, w2_ref[...], preferred_element_type=jnp.float32)

    @pl.when(i == 0)
    def _():
        u = _sp(jnp.dot(u0_ref[...], wa_ref[...], preferred_element_type=jnp.float32)
                + ba_ref[...])
        urow_ref[...] = u
        crow_ref[...] = jnp.dot(u, w4_ref[...], preferred_element_type=jnp.float32) + bce_ref[...]
        ucn_ref[...] = jnp.dot(u, wcnu_ref[...], preferred_element_type=jnp.float32) + bcn_ref[...]


def _prep_call(v0, Wn, bn, W1, W2, u0, Wa, ba, W4, bce, WcnU, bcn):
    full = pl.BlockSpec((D, D), lambda i: (0, 0))
    row = pl.BlockSpec((1, D), lambda i: (0, 0))
    blk = pl.BlockSpec((_BN, D), lambda i: (i, 0))
    return pl.pallas_call(
        _prep_body,
        grid=(_GN,),
        in_specs=[blk, full, row, full, full,
                  row, full, row, full, row, full, row],
        out_specs=[blk, blk, blk, row, row, row],
        out_shape=[
            jax.ShapeDtypeStruct((N, D), jnp.float32),
            jax.ShapeDtypeStruct((N, D), jnp.float32),
            jax.ShapeDtypeStruct((N, D), jnp.float32),
            jax.ShapeDtypeStruct((1, D), jnp.float32),
            jax.ShapeDtypeStruct((1, D), jnp.float32),
            jax.ShapeDtypeStruct((1, D), jnp.float32),
        ],
    )(v0, Wn, bn, W1, W2, u0, Wa, ba, W4, bce, WcnU, bcn)


_K = 5
_SUP = _NCH // _K


def _sc_gather_body(a_hbm, b_hbm, src_hbm, dst_hbm, g_hbm, cnt_hbm,
                    idx_s5, idx_d5, bufa5, bufb5, tab, semi, sema, semw):
    c = lax.axis_index("c")
    s = lax.axis_index("s")
    wid = s * _NC + c
    base = wid * _PER_W

    zero16 = jnp.zeros((16,), jnp.float32)
    one16 = jnp.ones((16,), jnp.float32)

    def zfill(r, carry):
        tab[pl.ds(r * 16, 16)] = zero16
        return carry

    lax.fori_loop(0, _NPAD // 16, zfill, 0)

    def issue_idx(t):
        off0 = pl.multiple_of(base + t * (_K * _CH), _CH)
        for k in range(_K):
            off = off0 + k * _CH
            pltpu.async_copy(src_hbm.at[pl.ds(off, _CH)], idx_s5.at[k], semi)
            pltpu.async_copy(dst_hbm.at[pl.ds(off, _CH)], idx_d5.at[k], semi)

    def wait_idx():
        for k in range(_K):
            pltpu.make_async_copy(src_hbm.at[pl.ds(base, _CH)], idx_s5.at[k], semi).wait()
            pltpu.make_async_copy(dst_hbm.at[pl.ds(base, _CH)], idx_d5.at[k], semi).wait()

    issue_idx(0)

    def body(t, carry):
        off0 = pl.multiple_of(base + t * (_K * _CH), _CH)
        wait_idx()
        gc = []
        for k in range(_K):
            gc.append(pltpu.async_copy(a_hbm.at[idx_s5.at[k]], bufa5.at[k], sema))
            gc.append(pltpu.async_copy(b_hbm.at[idx_d5.at[k]], bufb5.at[k], sema))
        wb = []
        for k in range(_K):
            gc[2 * k].wait()
            gc[2 * k + 1].wait()

            def addrow(r, carry, _k=k):
                for cc in range(D // 16):
                    sl = pl.ds(cc * 16, 16)
                    bufa5[_k, r, sl] += bufb5[_k, r, sl]
                return carry

            lax.fori_loop(0, _CH, addrow, 0)
            off = off0 + k * _CH
            wb.append(pltpu.async_copy(bufa5.at[k], g_hbm.at[pl.ds(off, _CH)], semw))
        for k in range(_K):
            for t16 in range(_CH // 16):
                ids = idx_d5[k, pl.ds(t16 * 16, 16)]
                plsc.addupdate_scatter(tab, [ids], one16)
        issue_idx(lax.rem(t + 1, _SUP))
        for cp in wb:
            cp.wait()
        return carry

    lax.fori_loop(0, _SUP, body, 0)
    wait_idx()
    pltpu.sync_copy(tab, cnt_hbm.at[wid])


@functools.partial(
    pl.kernel,
    out_type=[jax.ShapeDtypeStruct((E, D), jnp.float32),
              jax.ShapeDtypeStruct((_NW, _NPAD), jnp.float32)],
    mesh=plsc.VectorSubcoreMesh(core_axis_name="c", subcore_axis_name="s"),
    scratch_types=[
        pltpu.VMEM((_K, _CH), jnp.int32),
        pltpu.VMEM((_K, _CH), jnp.int32),
        pltpu.VMEM((_K, _CH, D), jnp.float32),
        pltpu.VMEM((_K, _CH, D), jnp.float32),
        pltpu.VMEM((_NPAD,), jnp.float32),
        pltpu.SemaphoreType.DMA,
        pltpu.SemaphoreType.DMA,
        pltpu.SemaphoreType.DMA,
    ],
    compiler_params=pltpu.CompilerParams(needs_layout_passes=False),
)
def _sc_gather(a_hbm, b_hbm, src_hbm, dst_hbm, g_hbm, cnt_hbm,
               idx_s5, idx_d5, bufa5, bufb5, tab, semi, sema, semw):
    _sc_gather_body(a_hbm, b_hbm, src_hbm, dst_hbm, g_hbm, cnt_hbm,
                    idx_s5, idx_d5, bufa5, bufb5, tab, semi, sema, semw)


def _edge_body(e0_ref, g_ref, we_ref, be_ref, w3_ref, crow_ref,
               oute_ref, enew_ref, ue_ref):
    i = pl.program_id(0)
    e0 = e0_ref[...]
    e = _sp(jnp.dot(e0, we_ref[...], preferred_element_type=jnp.float32) + be_ref[...])
    t = jnp.dot(e, w3_ref[...], preferred_element_type=jnp.float32)
    en = _sp(t + g_ref[...] + crow_ref[...])
    oute_ref[...] = en + e0
    enew_ref[...] = en
    part = jnp.sum(en.reshape(_BE // 8, 8, D), axis=0)

    @pl.when(i == 0)
    def _():
        ue_ref[...] = part

    @pl.when(i > 0)
    def _():
        ue_ref[...] += part


def _edge_call(e0, G, We, be, W3, crow):
    blk = pl.BlockSpec((_BE, D), lambda i: (i, 0))
    full = pl.BlockSpec((D, D), lambda i: (0, 0))
    row = pl.BlockSpec((1, D), lambda i: (0, 0))
    return pl.pallas_call(
        _edge_body,
        grid=(_GE,),
        in_specs=[blk, blk, full, row, full, row],
        out_specs=[blk, blk, pl.BlockSpec((8, D), lambda i: (0, 0))],
        out_shape=[
            jax.ShapeDtypeStruct((E, D), jnp.float32),
            jax.ShapeDtypeStruct((E, D), jnp.float32),
            jax.ShapeDtypeStruct((8, D), jnp.float32),
        ],
    )(e0, G, We, be, W3, crow)


_SSK = 1
_SCC = 80
_SCH = _SSK * _SCC
_SSUP = _PER_W // _SCH


def _sc_scatter_body(enew_hbm, dst_hbm, esum_hbm,
                     idx0, idx1, rows0, rows1, acc,
                     seml0, seml1, sems):
    c = lax.axis_index("c")
    s = lax.axis_index("s")
    wid = s * _NC + c
    base = wid * _PER_W

    zero16 = jnp.zeros((16,), jnp.float32)

    def zfill(r, carry):
        for cc in range(D // 16):
            rows0[r, pl.ds(cc * 16, 16)] = zero16
        return carry

    lax.fori_loop(0, _SCH, zfill, 0)
    for k in range(_ROWS_PER_TILE // _SCH):
        r0 = s * _ROWS_PER_TILE + k * _SCH
        pltpu.sync_copy(rows0, acc.at[pl.ds(r0, _SCH)])
    plsc.subcore_barrier()

    def issue_loads(sc, idx_b, rows_b, sem):
        off0 = pl.multiple_of(base + sc * _SCH, _SCC)
        for k in range(_SSK):
            pltpu.async_copy(dst_hbm.at[pl.ds(off0 + k * _SCC, _SCC)],
                             idx_b.at[k], sem)
        pltpu.async_copy(enew_hbm.at[pl.ds(off0, _SCH)], rows_b, sem)

    def wait_loads(idx_b, rows_b, sem):
        for k in range(_SSK):
            pltpu.make_async_copy(dst_hbm.at[pl.ds(base, _SCC)],
                                  idx_b.at[k], sem).wait()
        pltpu.make_async_copy(enew_hbm.at[pl.ds(base, _SCH)], rows_b, sem).wait()

    def do_scatter(idx_b, rows_b):
        cps = []
        for k in range(_SSK):
            cps.append(pltpu.async_copy(rows_b.at[pl.ds(k * _SCC, _SCC)],
                                        acc.at[idx_b.at[k]], sems, add=True))
        for cp in cps:
            cp.wait()

    issue_loads(0, idx0, rows0, seml0)

    def body(i, carry):
        issue_loads(2 * i + 1, idx1, rows1, seml1)
        wait_loads(idx0, rows0, seml0)
        do_scatter(idx0, rows0)
        issue_loads(2 * i + 2, idx0, rows0, seml0)
        wait_loads(idx1, rows1, seml1)
        do_scatter(idx1, rows1)
        return carry

    lax.fori_loop(0, (_SSUP - 1) // 2, body, 0)
    wait_loads(idx0, rows0, seml0)
    do_scatter(idx0, rows0)
    plsc.subcore_barrier()

    for k in range(_ROWS_PER_TILE // _ZCH):
        r0 = s * _ROWS_PER_TILE + k * _ZCH
        pltpu.sync_copy(acc.at[pl.ds(r0, _ZCH)], esum_hbm.at[c, pl.ds(r0, _ZCH)])


@functools.partial(
    pl.kernel,
    out_type=jax.ShapeDtypeStruct((_NC, _NPAD, D), jnp.float32),
    mesh=plsc.VectorSubcoreMesh(core_axis_name="c", subcore_axis_name="s"),
    scratch_types=[
        pltpu.VMEM((_SSK, _SCC), jnp.int32),
        pltpu.VMEM((_SSK, _SCC), jnp.int32),
        pltpu.VMEM((_SCH, D), jnp.float32),
        pltpu.VMEM((_SCH, D), jnp.float32),
        pltpu.VMEM_SHARED((_NPAD, D), jnp.float32),
        pltpu.SemaphoreType.DMA,
        pltpu.SemaphoreType.DMA,
        pltpu.SemaphoreType.DMA,
    ],
)
def _sc_scatter(enew_hbm, dst_hbm, esum_hbm,
                idx0, idx1, rows0, rows1, acc, seml0, seml1, sems):
    _sc_scatter_body(enew_hbm, dst_hbm, esum_hbm,
                     idx0, idx1, rows0, rows1, acc, seml0, seml1, sems)


def _node_body(v_ref, v0_ref, es0_ref, es1_ref, cnt_ref,
               ucn_ref, wv_ref, wve_ref,
               urow_ref, ue_ref, wa1_ref, wa2_ref, wa3_ref, bca_ref, u0_ref,
               outv_ref, outu_ref, uvacc_ref):
    i = pl.program_id(0)
    es = es0_ref[...] + es1_ref[...]
    cnt = jnp.sum(jnp.transpose(cnt_ref[...]), axis=1, keepdims=True)
    ve = es / jnp.maximum(cnt, 1.0)
    vn = _sp(jnp.dot(v_ref[...], wv_ref[...], preferred_element_type=jnp.float32)
             + jnp.dot(ve, wve_ref[...], preferred_element_type=jnp.float32)
             + ucn_ref[...])
    outv_ref[...] = vn + v0_ref[...]
    rows = i * _BN + lax.broadcasted_iota(jnp.int32, (_BN, 1), 0)
    vn_masked = jnp.where(rows < N, vn, 0.0)
    part = jnp.sum(vn_masked.reshape(_BN // 8, 8, D), axis=0)

    @pl.when(i == 0)
    def _():
        uvacc_ref[...] = part

    @pl.when(i > 0)
    def _():
        uvacc_ref[...] += part

    @pl.when(i == _GN - 1)
    def _():
        uv = jnp.sum(uvacc_ref[...], axis=0, keepdims=True) * (1.0 / N)
        ue = jnp.sum(ue_ref[...], axis=0, keepdims=True) * (1.0 / E)
        un = _sp(jnp.dot(urow_ref[...], wa1_ref[...], preferred_element_type=jnp.float32)
                 + jnp.dot(ue, wa2_ref[...], preferred_element_type=jnp.float32)
                 + jnp.dot(uv, wa3_ref[...], preferred_element_type=jnp.float32)
                 + bca_ref[...])
        outu_ref[...] = un + u0_ref[...]


def _node_call(v, v0, es0, es1, cnt_all, ucn, WcnV, WcnE,
               urow, ue_part, Wa1, Wa2, Wa3, bca, u0):
    blk = pl.BlockSpec((_BN, D), lambda i: (i, 0))
    cblk = pl.BlockSpec((_NW, _BN), lambda i: (0, i))
    full = pl.BlockSpec((D, D), lambda i: (0, 0))
    row = pl.BlockSpec((1, D), lambda i: (0, 0))
    return pl.pallas_call(
        _node_body,
        grid=(_GN,),
        in_specs=[blk, blk, blk, blk, cblk,
                  row, full, full,
                  row, pl.BlockSpec((8, D), lambda i: (0, 0)),
                  full, full, full, row, row],
        out_specs=[blk, row],
        out_shape=[
            jax.ShapeDtypeStruct((N, D), jnp.float32),
            jax.ShapeDtypeStruct((1, D), jnp.float32),
        ],
        scratch_shapes=[pltpu.VMEM((8, D), jnp.float32)],
    )(v, v0, es0, es1, cnt_all, ucn, WcnV, WcnE,
      urow, ue_part, Wa1, Wa2, Wa3, bca, u0)


def kernel(edge_feat, node_feat, graph_attr, edge_index,
           We, be, Wn, bn, Wa, ba, Wce, bce, Wcn, bcn, Wca, bca):
    src = edge_index[0]
    dst = edge_index[1]
    be_r = be.reshape(1, D)
    bn_r = bn.reshape(1, D)
    ba_r = ba.reshape(1, D)
    bce_r = bce.reshape(1, D)
    bcn_r = bcn.reshape(1, D)
    bca_r = bca.reshape(1, D)
    W1, W2, W3, W4 = Wce[:D], Wce[D:2 * D], Wce[2 * D:3 * D], Wce[3 * D:]
    WcnV, WcnE, WcnU = Wcn[:D], Wcn[D:2 * D], Wcn[2 * D:]
    Wa1, Wa2, Wa3 = Wca[:D], Wca[D:2 * D], Wca[2 * D:]

    v, A, B, crow, ucn, urow = _prep_call(
        node_feat, Wn, bn_r, W1, W2, graph_attr, Wa, ba_r, W4, bce_r, WcnU, bcn_r)

    G, cnt_all = _sc_gather(A, B, src, dst)

    out_e, e_new, ue_part = _edge_call(edge_feat, G, We, be_r, W3, crow)

    esum_part = _sc_scatter(e_new, dst)

    out_v, out_u = _node_call(
        v, node_feat, esum_part[0], esum_part[1], cnt_all,
        ucn, WcnV, WcnE, urow, ue_part, Wa1, Wa2, Wa3, bca_r, graph_attr)

    return (out_e, out_v, out_u)


# speedup vs baseline: 5.4334x; 1.0185x over previous
import functools

import jax
import jax.numpy as jnp
from jax import lax
from jax.experimental import pallas as pl
from jax.experimental.pallas import tpu as pltpu
from jax.experimental.pallas import tpu_sc as plsc

N = 10000
E = 320000
D = 128

_NC = 2
_NS = 16
_NW = _NC * _NS
_PER_W = E // _NW
_CH = 80
_NCH = _PER_W // _CH
_NPAD = 10240
_ROWS_PER_TILE = _NPAD // _NS
_ZCH = 128

_BN = 1024
_GN = _NPAD // _BN
_BE = 2560
_GE = E // _BE

_sp = jax.nn.softplus


def _prep_body(v0_ref, wn_ref, bn_ref, w1_ref, w2_ref,
               u0_ref, wa_ref, ba_ref, w4_ref, bce_ref, wcnu_ref, bcn_ref,
               v_ref, a_ref, b_ref, crow_ref, ucn_ref, urow_ref):
    i = pl.program_id(0)
    v = _sp(jnp.dot(v0_ref[...], wn_ref[...], preferred_element_type=jnp.float32)
            + bn_ref[...])
    v_ref[...] = v
    a_ref[...] = jnp.dot(v, w1_ref[...], preferred_element_type=jnp.float32)
    b_ref[...] = jnp.dot(v, w2_ref[...], preferred_element_type=jnp.float32)

    @pl.when(i == 0)
    def _():
        u = _sp(jnp.dot(u0_ref[...], wa_ref[...], preferred_element_type=jnp.float32)
                + ba_ref[...])
        urow_ref[...] = u
        crow_ref[...] = jnp.dot(u, w4_ref[...], preferred_element_type=jnp.float32) + bce_ref[...]
        ucn_ref[...] = jnp.dot(u, wcnu_ref[...], preferred_element_type=jnp.float32) + bcn_ref[...]


def _prep_call(v0, Wn, bn, W1, W2, u0, Wa, ba, W4, bce, WcnU, bcn):
    full = pl.BlockSpec((D, D), lambda i: (0, 0))
    row = pl.BlockSpec((1, D), lambda i: (0, 0))
    blk = pl.BlockSpec((_BN, D), lambda i: (i, 0))
    return pl.pallas_call(
        _prep_body,
        grid=(_GN,),
        in_specs=[blk, full, row, full, full,
                  row, full, row, full, row, full, row],
        out_specs=[blk, blk, blk, row, row, row],
        out_shape=[
            jax.ShapeDtypeStruct((N, D), jnp.float32),
            jax.ShapeDtypeStruct((N, D), jnp.float32),
            jax.ShapeDtypeStruct((N, D), jnp.float32),
            jax.ShapeDtypeStruct((1, D), jnp.float32),
            jax.ShapeDtypeStruct((1, D), jnp.float32),
            jax.ShapeDtypeStruct((1, D), jnp.float32),
        ],
    )(v0, Wn, bn, W1, W2, u0, Wa, ba, W4, bce, WcnU, bcn)


_K = 5
_SUP = _NCH // _K


def _sc_gather_body(a_hbm, b_hbm, src_hbm, dst_hbm, g_hbm, cnt_hbm,
                    idx_s5, idx_d5, bufa5, bufb5, tab, semi, sema, semw):
    c = lax.axis_index("c")
    s = lax.axis_index("s")
    wid = s * _NC + c
    base = wid * _PER_W

    zero16 = jnp.zeros((16,), jnp.float32)
    one16 = jnp.ones((16,), jnp.float32)

    def zfill(r, carry):
        tab[pl.ds(r * 16, 16)] = zero16
        return carry

    lax.fori_loop(0, _NPAD // 16, zfill, 0)

    def issue_idx(t):
        off0 = pl.multiple_of(base + t * (_K * _CH), _CH)
        for k in range(_K):
            off = off0 + k * _CH
            pltpu.async_copy(src_hbm.at[pl.ds(off, _CH)], idx_s5.at[k], semi)
            pltpu.async_copy(dst_hbm.at[pl.ds(off, _CH)], idx_d5.at[k], semi)

    def wait_idx():
        for k in range(_K):
            pltpu.make_async_copy(src_hbm.at[pl.ds(base, _CH)], idx_s5.at[k], semi).wait()
            pltpu.make_async_copy(dst_hbm.at[pl.ds(base, _CH)], idx_d5.at[k], semi).wait()

    issue_idx(0)

    def body(t, carry):
        off0 = pl.multiple_of(base + t * (_K * _CH), _CH)
        wait_idx()
        gc = []
        for k in range(_K):
            gc.append(pltpu.async_copy(a_hbm.at[idx_s5.at[k]], bufa5.at[k], sema))
            gc.append(pltpu.async_copy(b_hbm.at[idx_d5.at[k]], bufb5.at[k], sema))
        wb = []
        for k in range(_K):
            gc[2 * k].wait()
            gc[2 * k + 1].wait()

            def addrow(r, carry, _k=k):
                for cc in range(D // 16):
                    sl = pl.ds(cc * 16, 16)
                    bufa5[_k, r, sl] += bufb5[_k, r, sl]
                return carry

            lax.fori_loop(0, _CH, addrow, 0)
            off = off0 + k * _CH
            wb.append(pltpu.async_copy(bufa5.at[k], g_hbm.at[pl.ds(off, _CH)], semw))
        for k in range(_K):
            for t16 in range(_CH // 16):
                ids = idx_d5[k, pl.ds(t16 * 16, 16)]
                plsc.addupdate_scatter(tab, [ids], one16)
        issue_idx(lax.rem(t + 1, _SUP))
        for cp in wb:
            cp.wait()
        return carry

    lax.fori_loop(0, _SUP, body, 0)
    wait_idx()
    pltpu.sync_copy(tab, cnt_hbm.at[wid])


@functools.partial(
    pl.kernel,
    out_type=[jax.ShapeDtypeStruct((E, D), jnp.float32),
              jax.ShapeDtypeStruct((_NW, _NPAD), jnp.float32)],
    mesh=plsc.VectorSubcoreMesh(core_axis_name="c", subcore_axis_name="s"),
    scratch_types=[
        pltpu.VMEM((_K, _CH), jnp.int32),
        pltpu.VMEM((_K, _CH), jnp.int32),
        pltpu.VMEM((_K, _CH, D), jnp.float32),
        pltpu.VMEM((_K, _CH, D), jnp.float32),
        pltpu.VMEM((_NPAD,), jnp.float32),
        pltpu.SemaphoreType.DMA,
        pltpu.SemaphoreType.DMA,
        pltpu.SemaphoreType.DMA,
    ],
    compiler_params=pltpu.CompilerParams(needs_layout_passes=False),
)
def _sc_gather(a_hbm, b_hbm, src_hbm, dst_hbm, g_hbm, cnt_hbm,
               idx_s5, idx_d5, bufa5, bufb5, tab, semi, sema, semw):
    _sc_gather_body(a_hbm, b_hbm, src_hbm, dst_hbm, g_hbm, cnt_hbm,
                    idx_s5, idx_d5, bufa5, bufb5, tab, semi, sema, semw)


def _splean(x):
    return jnp.maximum(x, 0.0) + jnp.log1p(jnp.exp(-jnp.abs(x)))


def _edge_body(e0_ref, g_ref, we_ref, be_ref, w3_ref, crow_ref,
               oute_ref, enew_ref):
    e0 = e0_ref[...]
    e = _splean(jnp.dot(e0, we_ref[...], preferred_element_type=jnp.float32) + be_ref[...])
    t = jnp.dot(e, w3_ref[...], preferred_element_type=jnp.float32)
    en = _splean(t + g_ref[...] + crow_ref[...])
    oute_ref[...] = en + e0
    enew_ref[...] = en


def _edge_call(e0, G, We, be, W3, crow):
    blk = pl.BlockSpec((_BE, D), lambda i: (i, 0))
    full = pl.BlockSpec((D, D), lambda i: (0, 0))
    row = pl.BlockSpec((1, D), lambda i: (0, 0))
    return pl.pallas_call(
        _edge_body,
        grid=(_GE,),
        in_specs=[blk, blk, full, row, full, row],
        out_specs=[blk, blk],
        out_shape=[
            jax.ShapeDtypeStruct((E, D), jnp.float32),
            jax.ShapeDtypeStruct((E, D), jnp.float32),
        ],
    )(e0, G, We, be, W3, crow)


_SSK = 1
_SCC = 80
_SCH = _SSK * _SCC
_SSUP = _PER_W // _SCH


def _sc_scatter_body(enew_hbm, dst_hbm, esum_hbm,
                     idx0, idx1, rows0, rows1, acc,
                     seml0, seml1, sems):
    c = lax.axis_index("c")
    s = lax.axis_index("s")
    wid = s * _NC + c
    base = wid * _PER_W

    zero16 = jnp.zeros((16,), jnp.float32)

    def zfill(r, carry):
        for cc in range(D // 16):
            rows0[r, pl.ds(cc * 16, 16)] = zero16
        return carry

    lax.fori_loop(0, _SCH, zfill, 0)
    for k in range(_ROWS_PER_TILE // _SCH):
        r0 = s * _ROWS_PER_TILE + k * _SCH
        pltpu.sync_copy(rows0, acc.at[pl.ds(r0, _SCH)])
    plsc.subcore_barrier()

    def issue_loads(sc, idx_b, rows_b, sem):
        off0 = pl.multiple_of(base + sc * _SCH, _SCC)
        for k in range(_SSK):
            pltpu.async_copy(dst_hbm.at[pl.ds(off0 + k * _SCC, _SCC)],
                             idx_b.at[k], sem)
        pltpu.async_copy(enew_hbm.at[pl.ds(off0, _SCH)], rows_b, sem)

    def wait_loads(idx_b, rows_b, sem):
        for k in range(_SSK):
            pltpu.make_async_copy(dst_hbm.at[pl.ds(base, _SCC)],
                                  idx_b.at[k], sem).wait()
        pltpu.make_async_copy(enew_hbm.at[pl.ds(base, _SCH)], rows_b, sem).wait()

    def do_scatter(idx_b, rows_b):
        cps = []
        for k in range(_SSK):
            cps.append(pltpu.async_copy(rows_b.at[pl.ds(k * _SCC, _SCC)],
                                        acc.at[idx_b.at[k]], sems, add=True))
        for cp in cps:
            cp.wait()

    issue_loads(0, idx0, rows0, seml0)

    def body(i, carry):
        issue_loads(2 * i + 1, idx1, rows1, seml1)
        wait_loads(idx0, rows0, seml0)
        do_scatter(idx0, rows0)
        issue_loads(2 * i + 2, idx0, rows0, seml0)
        wait_loads(idx1, rows1, seml1)
        do_scatter(idx1, rows1)
        return carry

    lax.fori_loop(0, (_SSUP - 1) // 2, body, 0)
    wait_loads(idx0, rows0, seml0)
    do_scatter(idx0, rows0)
    plsc.subcore_barrier()

    for k in range(_ROWS_PER_TILE // _ZCH):
        r0 = s * _ROWS_PER_TILE + k * _ZCH
        pltpu.sync_copy(acc.at[pl.ds(r0, _ZCH)], esum_hbm.at[c, pl.ds(r0, _ZCH)])


@functools.partial(
    pl.kernel,
    out_type=jax.ShapeDtypeStruct((_NC, _NPAD, D), jnp.float32),
    mesh=plsc.VectorSubcoreMesh(core_axis_name="c", subcore_axis_name="s"),
    scratch_types=[
        pltpu.VMEM((_SSK, _SCC), jnp.int32),
        pltpu.VMEM((_SSK, _SCC), jnp.int32),
        pltpu.VMEM((_SCH, D), jnp.float32),
        pltpu.VMEM((_SCH, D), jnp.float32),
        pltpu.VMEM_SHARED((_NPAD, D), jnp.float32),
        pltpu.SemaphoreType.DMA,
        pltpu.SemaphoreType.DMA,
        pltpu.SemaphoreType.DMA,
    ],
)
def _sc_scatter(enew_hbm, dst_hbm, esum_hbm,
                idx0, idx1, rows0, rows1, acc, seml0, seml1, sems):
    _sc_scatter_body(enew_hbm, dst_hbm, esum_hbm,
                     idx0, idx1, rows0, rows1, acc, seml0, seml1, sems)


def _node_body(v_ref, v0_ref, es0_ref, es1_ref, cnt_ref,
               ucn_ref, wv_ref, wve_ref,
               urow_ref, wa1_ref, wa2_ref, wa3_ref, bca_ref, u0_ref,
               outv_ref, outu_ref, uvacc_ref, ueacc_ref):
    i = pl.program_id(0)
    es = es0_ref[...] + es1_ref[...]
    cnt = jnp.sum(jnp.transpose(cnt_ref[...]), axis=1, keepdims=True)
    ve = es / jnp.maximum(cnt, 1.0)
    vn = _sp(jnp.dot(v_ref[...], wv_ref[...], preferred_element_type=jnp.float32)
             + jnp.dot(ve, wve_ref[...], preferred_element_type=jnp.float32)
             + ucn_ref[...])
    outv_ref[...] = vn + v0_ref[...]
    rows = i * _BN + lax.broadcasted_iota(jnp.int32, (_BN, 1), 0)
    vn_masked = jnp.where(rows < N, vn, 0.0)
    part = jnp.sum(vn_masked.reshape(_BN // 8, 8, D), axis=0)
    epart = jnp.sum(es.reshape(_BN // 8, 8, D), axis=0)

    @pl.when(i == 0)
    def _():
        uvacc_ref[...] = part
        ueacc_ref[...] = epart

    @pl.when(i > 0)
    def _():
        uvacc_ref[...] += part
        ueacc_ref[...] += epart

    @pl.when(i == _GN - 1)
    def _():
        uv = jnp.sum(uvacc_ref[...], axis=0, keepdims=True) * (1.0 / N)
        ue = jnp.sum(ueacc_ref[...], axis=0, keepdims=True) * (1.0 / E)
        un = _sp(jnp.dot(urow_ref[...], wa1_ref[...], preferred_element_type=jnp.float32)
                 + jnp.dot(ue, wa2_ref[...], preferred_element_type=jnp.float32)
                 + jnp.dot(uv, wa3_ref[...], preferred_element_type=jnp.float32)
                 + bca_ref[...])
        outu_ref[...] = un + u0_ref[...]


def _node_call(v, v0, es0, es1, cnt_all, ucn, WcnV, WcnE,
               urow, Wa1, Wa2, Wa3, bca, u0):
    blk = pl.BlockSpec((_BN, D), lambda i: (i, 0))
    cblk = pl.BlockSpec((_NW, _BN), lambda i: (0, i))
    full = pl.BlockSpec((D, D), lambda i: (0, 0))
    row = pl.BlockSpec((1, D), lambda i: (0, 0))
    return pl.pallas_call(
        _node_body,
        grid=(_GN,),
        in_specs=[blk, blk, blk, blk, cblk,
                  row, full, full,
                  row,
                  full, full, full, row, row],
        out_specs=[blk, row],
        out_shape=[
            jax.ShapeDtypeStruct((N, D), jnp.float32),
            jax.ShapeDtypeStruct((1, D), jnp.float32),
        ],
        scratch_shapes=[pltpu.VMEM((8, D), jnp.float32),
                        pltpu.VMEM((8, D), jnp.float32)],
    )(v, v0, es0, es1, cnt_all, ucn, WcnV, WcnE,
      urow, Wa1, Wa2, Wa3, bca, u0)


def kernel(edge_feat, node_feat, graph_attr, edge_index,
           We, be, Wn, bn, Wa, ba, Wce, bce, Wcn, bcn, Wca, bca):
    src = edge_index[0]
    dst = edge_index[1]
    be_r = be.reshape(1, D)
    bn_r = bn.reshape(1, D)
    ba_r = ba.reshape(1, D)
    bce_r = bce.reshape(1, D)
    bcn_r = bcn.reshape(1, D)
    bca_r = bca.reshape(1, D)
    W1, W2, W3, W4 = Wce[:D], Wce[D:2 * D], Wce[2 * D:3 * D], Wce[3 * D:]
    WcnV, WcnE, WcnU = Wcn[:D], Wcn[D:2 * D], Wcn[2 * D:]
    Wa1, Wa2, Wa3 = Wca[:D], Wca[D:2 * D], Wca[2 * D:]

    v, A, B, crow, ucn, urow = _prep_call(
        node_feat, Wn, bn_r, W1, W2, graph_attr, Wa, ba_r, W4, bce_r, WcnU, bcn_r)

    G, cnt_all = _sc_gather(A, B, src, dst)

    out_e, e_new = _edge_call(edge_feat, G, We, be_r, W3, crow)

    esum_part = _sc_scatter(e_new, dst)

    out_v, out_u = _node_call(
        v, node_feat, esum_part[0], esum_part[1], cnt_all,
        ucn, WcnV, WcnE, urow, Wa1, Wa2, Wa3, bca_r, graph_attr)

    return (out_e, out_v, out_u)


# speedup vs baseline: 5.4371x; 1.0007x over previous
import functools

import jax
import jax.numpy as jnp
from jax import lax
from jax.experimental import pallas as pl
from jax.experimental.pallas import tpu as pltpu
from jax.experimental.pallas import tpu_sc as plsc

N = 10000
E = 320000
D = 128

_NC = 2
_NS = 16
_NW = _NC * _NS
_PER_W = E // _NW
_CH = 80
_NCH = _PER_W // _CH
_NPAD = 10240
_ROWS_PER_TILE = _NPAD // _NS
_ZCH = 128

_BN = 1024
_GN = _NPAD // _BN
_BE = 2560
_GE = E // _BE

_sp = jax.nn.softplus


def _prep_body(v0_ref, wn_ref, bn_ref, w1_ref, w2_ref,
               u0_ref, wa_ref, ba_ref, w4_ref, bce_ref, wcnu_ref, bcn_ref,
               v_ref, a_ref, b_ref, crow_ref, ucn_ref, urow_ref):
    i = pl.program_id(0)
    v = _sp(jnp.dot(v0_ref[...], wn_ref[...], preferred_element_type=jnp.float32)
            + bn_ref[...])
    v_ref[...] = v
    a_ref[...] = jnp.dot(v, w1_ref[...], preferred_element_type=jnp.float32)
    b_ref[...] = jnp.dot(v, w2_ref[...], preferred_element_type=jnp.float32)

    @pl.when(i == 0)
    def _():
        u = _sp(jnp.dot(u0_ref[...], wa_ref[...], preferred_element_type=jnp.float32)
                + ba_ref[...])
        urow_ref[...] = u
        crow_ref[...] = jnp.dot(u, w4_ref[...], preferred_element_type=jnp.float32) + bce_ref[...]
        ucn_ref[...] = jnp.dot(u, wcnu_ref[...], preferred_element_type=jnp.float32) + bcn_ref[...]


def _prep_call(v0, Wn, bn, W1, W2, u0, Wa, ba, W4, bce, WcnU, bcn):
    full = pl.BlockSpec((D, D), lambda i: (0, 0))
    row = pl.BlockSpec((1, D), lambda i: (0, 0))
    blk = pl.BlockSpec((_BN, D), lambda i: (i, 0))
    return pl.pallas_call(
        _prep_body,
        grid=(_GN,),
        in_specs=[blk, full, row, full, full,
                  row, full, row, full, row, full, row],
        out_specs=[blk, blk, blk, row, row, row],
        out_shape=[
            jax.ShapeDtypeStruct((N, D), jnp.float32),
            jax.ShapeDtypeStruct((N, D), jnp.float32),
            jax.ShapeDtypeStruct((N, D), jnp.float32),
            jax.ShapeDtypeStruct((1, D), jnp.float32),
            jax.ShapeDtypeStruct((1, D), jnp.float32),
            jax.ShapeDtypeStruct((1, D), jnp.float32),
        ],
    )(v0, Wn, bn, W1, W2, u0, Wa, ba, W4, bce, WcnU, bcn)


_K = 5
_SUP = _NCH // _K


def _sc_gather_body(a_hbm, b_hbm, src_hbm, dst_hbm, g_hbm, cnt_hbm,
                    idx_s5, idx_d5, bufa5, bufb5, tab, semi, sema, semw):
    c = lax.axis_index("c")
    s = lax.axis_index("s")
    wid = s * _NC + c
    base = wid * _PER_W

    zero16 = jnp.zeros((16,), jnp.float32)
    one16 = jnp.ones((16,), jnp.float32)

    def zfill(r, carry):
        tab[pl.ds(r * 16, 16)] = zero16
        return carry

    lax.fori_loop(0, _NPAD // 16, zfill, 0)

    def issue_idx(t):
        off0 = pl.multiple_of(base + t * (_K * _CH), _CH)
        for k in range(_K):
            off = off0 + k * _CH
            pltpu.async_copy(src_hbm.at[pl.ds(off, _CH)], idx_s5.at[k], semi)
            pltpu.async_copy(dst_hbm.at[pl.ds(off, _CH)], idx_d5.at[k], semi)

    def wait_idx():
        for k in range(_K):
            pltpu.make_async_copy(src_hbm.at[pl.ds(base, _CH)], idx_s5.at[k], semi).wait()
            pltpu.make_async_copy(dst_hbm.at[pl.ds(base, _CH)], idx_d5.at[k], semi).wait()

    issue_idx(0)

    def body(t, carry):
        off0 = pl.multiple_of(base + t * (_K * _CH), _CH)
        wait_idx()
        gc = []
        for k in range(_K):
            gc.append(pltpu.async_copy(a_hbm.at[idx_s5.at[k]], bufa5.at[k], sema))
            gc.append(pltpu.async_copy(b_hbm.at[idx_d5.at[k]], bufb5.at[k], sema))
        wb = []
        for k in range(_K):
            gc[2 * k].wait()
            gc[2 * k + 1].wait()

            def addrow(r, carry, _k=k):
                for cc in range(D // 16):
                    sl = pl.ds(cc * 16, 16)
                    bufa5[_k, r, sl] += bufb5[_k, r, sl]
                return carry

            lax.fori_loop(0, _CH, addrow, 0)
            off = off0 + k * _CH
            wb.append(pltpu.async_copy(bufa5.at[k], g_hbm.at[pl.ds(off, _CH)], semw))
        for k in range(_K):
            for t16 in range(_CH // 16):
                ids = idx_d5[k, pl.ds(t16 * 16, 16)]
                plsc.addupdate_scatter(tab, [ids], one16)
        issue_idx(lax.rem(t + 1, _SUP))
        for cp in wb:
            cp.wait()
        return carry

    lax.fori_loop(0, _SUP, body, 0)
    wait_idx()
    pltpu.sync_copy(tab, cnt_hbm.at[wid])


@functools.partial(
    pl.kernel,
    out_type=[jax.ShapeDtypeStruct((E, D), jnp.float32),
              jax.ShapeDtypeStruct((_NW, _NPAD), jnp.float32)],
    mesh=plsc.VectorSubcoreMesh(core_axis_name="c", subcore_axis_name="s"),
    scratch_types=[
        pltpu.VMEM((_K, _CH), jnp.int32),
        pltpu.VMEM((_K, _CH), jnp.int32),
        pltpu.VMEM((_K, _CH, D), jnp.float32),
        pltpu.VMEM((_K, _CH, D), jnp.float32),
        pltpu.VMEM((_NPAD,), jnp.float32),
        pltpu.SemaphoreType.DMA,
        pltpu.SemaphoreType.DMA,
        pltpu.SemaphoreType.DMA,
    ],
    compiler_params=pltpu.CompilerParams(needs_layout_passes=False),
)
def _sc_gather(a_hbm, b_hbm, src_hbm, dst_hbm, g_hbm, cnt_hbm,
               idx_s5, idx_d5, bufa5, bufb5, tab, semi, sema, semw):
    _sc_gather_body(a_hbm, b_hbm, src_hbm, dst_hbm, g_hbm, cnt_hbm,
                    idx_s5, idx_d5, bufa5, bufb5, tab, semi, sema, semw)


def _splean(x):
    return jnp.maximum(x, 0.0) + jnp.log1p(jnp.exp(-jnp.abs(x)))


def _edge_body(e0_ref, g_ref, we_ref, be_ref, w3_ref, crow_ref,
               oute_ref, enew_ref):
    e0 = e0_ref[...]
    e = _splean(jnp.dot(e0, we_ref[...], preferred_element_type=jnp.float32) + be_ref[...])
    t = jnp.dot(e, w3_ref[...], preferred_element_type=jnp.float32)
    en = _splean(t + g_ref[...] + crow_ref[...])
    oute_ref[...] = en + e0
    enew_ref[...] = en


def _edge_call(e0, G, We, be, W3, crow):
    blk = pl.BlockSpec((_BE, D), lambda i: (i, 0))
    full = pl.BlockSpec((D, D), lambda i: (0, 0))
    row = pl.BlockSpec((1, D), lambda i: (0, 0))
    return pl.pallas_call(
        _edge_body,
        grid=(_GE,),
        in_specs=[blk, blk, full, row, full, row],
        out_specs=[blk, blk],
        out_shape=[
            jax.ShapeDtypeStruct((E, D), jnp.float32),
            jax.ShapeDtypeStruct((E, D), jnp.float32),
        ],
    )(e0, G, We, be, W3, crow)


_SSK = 1
_SCC = 80
_SCH = _SSK * _SCC
_SSUP = _PER_W // _SCH


def _sc_scatter_body(enew_hbm, dst_hbm, esum_hbm,
                     idx0, idx1, rows0, rows1, acc,
                     seml0, seml1, sems):
    c = lax.axis_index("c")
    s = lax.axis_index("s")
    wid = s * _NC + c
    base = wid * _PER_W

    zero16 = jnp.zeros((16,), jnp.float32)

    def zfill(r, carry):
        for cc in range(D // 16):
            rows0[r, pl.ds(cc * 16, 16)] = zero16
        return carry

    lax.fori_loop(0, _SCH, zfill, 0)
    zc = []
    for k in range(_ROWS_PER_TILE // _SCH):
        r0 = s * _ROWS_PER_TILE + k * _SCH
        zc.append(pltpu.async_copy(rows0, acc.at[pl.ds(r0, _SCH)], sems))
    for cp in zc:
        cp.wait()
    plsc.subcore_barrier()

    def issue_loads(sc, idx_b, rows_b, sem):
        off0 = pl.multiple_of(base + sc * _SCH, _SCC)
        for k in range(_SSK):
            pltpu.async_copy(dst_hbm.at[pl.ds(off0 + k * _SCC, _SCC)],
                             idx_b.at[k], sem)
        pltpu.async_copy(enew_hbm.at[pl.ds(off0, _SCH)], rows_b, sem)

    def wait_loads(idx_b, rows_b, sem):
        for k in range(_SSK):
            pltpu.make_async_copy(dst_hbm.at[pl.ds(base, _SCC)],
                                  idx_b.at[k], sem).wait()
        pltpu.make_async_copy(enew_hbm.at[pl.ds(base, _SCH)], rows_b, sem).wait()

    def do_scatter(idx_b, rows_b):
        cps = []
        for k in range(_SSK):
            cps.append(pltpu.async_copy(rows_b.at[pl.ds(k * _SCC, _SCC)],
                                        acc.at[idx_b.at[k]], sems, add=True))
        for cp in cps:
            cp.wait()

    issue_loads(0, idx0, rows0, seml0)

    def body(i, carry):
        issue_loads(2 * i + 1, idx1, rows1, seml1)
        wait_loads(idx0, rows0, seml0)
        do_scatter(idx0, rows0)
        issue_loads(2 * i + 2, idx0, rows0, seml0)
        wait_loads(idx1, rows1, seml1)
        do_scatter(idx1, rows1)
        return carry

    lax.fori_loop(0, (_SSUP - 1) // 2, body, 0)
    wait_loads(idx0, rows0, seml0)
    do_scatter(idx0, rows0)
    plsc.subcore_barrier()

    rb = []
    for k in range(_ROWS_PER_TILE // _ZCH):
        r0 = s * _ROWS_PER_TILE + k * _ZCH
        rb.append(pltpu.async_copy(acc.at[pl.ds(r0, _ZCH)],
                                   esum_hbm.at[c, pl.ds(r0, _ZCH)], sems))
    for cp in rb:
        cp.wait()


@functools.partial(
    pl.kernel,
    out_type=jax.ShapeDtypeStruct((_NC, _NPAD, D), jnp.float32),
    mesh=plsc.VectorSubcoreMesh(core_axis_name="c", subcore_axis_name="s"),
    scratch_types=[
        pltpu.VMEM((_SSK, _SCC), jnp.int32),
        pltpu.VMEM((_SSK, _SCC), jnp.int32),
        pltpu.VMEM((_SCH, D), jnp.float32),
        pltpu.VMEM((_SCH, D), jnp.float32),
        pltpu.VMEM_SHARED((_NPAD, D), jnp.float32),
        pltpu.SemaphoreType.DMA,
        pltpu.SemaphoreType.DMA,
        pltpu.SemaphoreType.DMA,
    ],
)
def _sc_scatter(enew_hbm, dst_hbm, esum_hbm,
                idx0, idx1, rows0, rows1, acc, seml0, seml1, sems):
    _sc_scatter_body(enew_hbm, dst_hbm, esum_hbm,
                     idx0, idx1, rows0, rows1, acc, seml0, seml1, sems)


def _node_body(v_ref, v0_ref, es0_ref, es1_ref, cnt_ref,
               ucn_ref, wv_ref, wve_ref,
               urow_ref, wa1_ref, wa2_ref, wa3_ref, bca_ref, u0_ref,
               outv_ref, outu_ref, uvacc_ref, ueacc_ref):
    i = pl.program_id(0)
    es = es0_ref[...] + es1_ref[...]
    cnt = jnp.sum(jnp.transpose(cnt_ref[...]), axis=1, keepdims=True)
    ve = es / jnp.maximum(cnt, 1.0)
    vn = _sp(jnp.dot(v_ref[...], wv_ref[...], preferred_element_type=jnp.float32)
             + jnp.dot(ve, wve_ref[...], preferred_element_type=jnp.float32)
             + ucn_ref[...])
    outv_ref[...] = vn + v0_ref[...]
    rows = i * _BN + lax.broadcasted_iota(jnp.int32, (_BN, 1), 0)
    vn_masked = jnp.where(rows < N, vn, 0.0)
    part = jnp.sum(vn_masked.reshape(_BN // 8, 8, D), axis=0)
    epart = jnp.sum(es.reshape(_BN // 8, 8, D), axis=0)

    @pl.when(i == 0)
    def _():
        uvacc_ref[...] = part
        ueacc_ref[...] = epart

    @pl.when(i > 0)
    def _():
        uvacc_ref[...] += part
        ueacc_ref[...] += epart

    @pl.when(i == _GN - 1)
    def _():
        uv = jnp.sum(uvacc_ref[...], axis=0, keepdims=True) * (1.0 / N)
        ue = jnp.sum(ueacc_ref[...], axis=0, keepdims=True) * (1.0 / E)
        un = _sp(jnp.dot(urow_ref[...], wa1_ref[...], preferred_element_type=jnp.float32)
                 + jnp.dot(ue, wa2_ref[...], preferred_element_type=jnp.float32)
                 + jnp.dot(uv, wa3_ref[...], preferred_element_type=jnp.float32)
                 + bca_ref[...])
        outu_ref[...] = un + u0_ref[...]


def _node_call(v, v0, es0, es1, cnt_all, ucn, WcnV, WcnE,
               urow, Wa1, Wa2, Wa3, bca, u0):
    blk = pl.BlockSpec((_BN, D), lambda i: (i, 0))
    cblk = pl.BlockSpec((_NW, _BN), lambda i: (0, i))
    full = pl.BlockSpec((D, D), lambda i: (0, 0))
    row = pl.BlockSpec((1, D), lambda i: (0, 0))
    return pl.pallas_call(
        _node_body,
        grid=(_GN,),
        in_specs=[blk, blk, blk, blk, cblk,
                  row, full, full,
                  row,
                  full, full, full, row, row],
        out_specs=[blk, row],
        out_shape=[
            jax.ShapeDtypeStruct((N, D), jnp.float32),
            jax.ShapeDtypeStruct((1, D), jnp.float32),
        ],
        scratch_shapes=[pltpu.VMEM((8, D), jnp.float32),
                        pltpu.VMEM((8, D), jnp.float32)],
    )(v, v0, es0, es1, cnt_all, ucn, WcnV, WcnE,
      urow, Wa1, Wa2, Wa3, bca, u0)


def kernel(edge_feat, node_feat, graph_attr, edge_index,
           We, be, Wn, bn, Wa, ba, Wce, bce, Wcn, bcn, Wca, bca):
    src = edge_index[0]
    dst = edge_index[1]
    be_r = be.reshape(1, D)
    bn_r = bn.reshape(1, D)
    ba_r = ba.reshape(1, D)
    bce_r = bce.reshape(1, D)
    bcn_r = bcn.reshape(1, D)
    bca_r = bca.reshape(1, D)
    W1, W2, W3, W4 = Wce[:D], Wce[D:2 * D], Wce[2 * D:3 * D], Wce[3 * D:]
    WcnV, WcnE, WcnU = Wcn[:D], Wcn[D:2 * D], Wcn[2 * D:]
    Wa1, Wa2, Wa3 = Wca[:D], Wca[D:2 * D], Wca[2 * D:]

    v, A, B, crow, ucn, urow = _prep_call(
        node_feat, Wn, bn_r, W1, W2, graph_attr, Wa, ba_r, W4, bce_r, WcnU, bcn_r)

    G, cnt_all = _sc_gather(A, B, src, dst)

    out_e, e_new = _edge_call(edge_feat, G, We, be_r, W3, crow)

    esum_part = _sc_scatter(e_new, dst)

    out_v, out_u = _node_call(
        v, node_feat, esum_part[0], esum_part[1], cnt_all,
        ucn, WcnV, WcnE, urow, Wa1, Wa2, Wa3, bca_r, graph_attr)

    return (out_e, out_v, out_u)
